# Initial kernel scaffold; baseline (speedup 1.0000x reference)
import jax
import jax.numpy as jnp
from jax.experimental import pallas as pl


def kernel(src_img, target_img, src_mask, target_mask, ref_img):
    raise NotImplementedError("write your pallas kernel here")


# speedup vs baseline: 55.9128x; 55.9128x over previous
import jax
import jax.numpy as jnp
from jax import lax
from jax.experimental import pallas as pl
from jax.experimental.pallas import tpu as pltpu
from jax.experimental.pallas import tpu_sc as plsc

K = 512
NB = K + 1
L = 16
NBP = ((NB + L - 1) // L) * L + L
NCORES = 2
NSUB = 16
NCH = 12
CH = 4096
K_F = float(K)
INVK = 1.0 / K


def _body(ref2, tgt2, sm2, tm2, src2, out,
          hist_r, hist_t, cx_r, c_r, c_t, abuf, dbuf, b0, b1, b2, b3):
    npix = ref2.shape[1]
    nchunk = npix // CH
    nf = jnp.float32(npix)
    core = lax.axis_index("c")
    sub = lax.axis_index("s")
    ch = core * (NCH // NCORES) + sub
    b = ch // 3
    lane = lax.iota(jnp.int32, L)
    onesv = jnp.ones((L,), jnp.float32)

    def to_bin(x):
        kf = x * K_F
        ki = kf.astype(jnp.int32)
        ki = jnp.where(kf > ki.astype(jnp.float32), ki + 1, ki)
        return kf, ki

    @pl.when(sub < (NCH // NCORES))
    def _():
        def zero_body(i, carry):
            z = jnp.zeros((L,), jnp.float32)
            hist_r[pl.ds(i * L, L)] = z
            hist_t[pl.ds(i * L, L)] = z
            return carry
        lax.fori_loop(0, NBP, zero_body, 0)

        def chunk_a(cidx, carry):
            off = cidx * CH
            pltpu.sync_copy(ref2.at[ch, pl.ds(off, CH)], b0)
            pltpu.sync_copy(sm2.at[b, pl.ds(off, CH)], b1)
            pltpu.sync_copy(tgt2.at[ch, pl.ds(off, CH)], b2)
            pltpu.sync_copy(tm2.at[b, pl.ds(off, CH)], b3)

            def inner(i, carry2):
                sl = pl.ds(i * L, L)
                rv = b0[sl]
                mv = b1[sl]
                tv = b2[sl]
                mtv = b3[sl]
                r = jnp.minimum(jnp.maximum(rv * 0.5 + 0.5, 0.0), 1.0) * mv
                t = jnp.minimum(jnp.maximum(tv * 0.5 + 0.5, 0.0), 1.0) * mtv
                _, kir = to_bin(r)
                _, kit = to_bin(t)
                plsc.addupdate_scatter(hist_r, [kir * L + lane], onesv)
                plsc.addupdate_scatter(hist_t, [kit * L + lane], onesv)
                return carry2
            lax.fori_loop(0, CH // L, inner, 0)
            return carry
        lax.fori_loop(0, nchunk, chunk_a, 0)

        def a2(j, carries):
            car_r, car_t = carries
            binv = j * L + lane
            accr = jnp.zeros((L,), jnp.float32)
            acct = jnp.zeros((L,), jnp.float32)
            for l in range(L):
                accr = accr + plsc.load_gather(hist_r, [binv * L + l])
                acct = acct + plsc.load_gather(hist_t, [binv * L + l])
            incr = plsc.cumsum(accr) + car_r
            inct = plsc.cumsum(acct) + car_t
            sl = pl.ds(j * L, L)
            cx_r[sl] = incr - accr
            c_r[sl] = incr
            c_t[sl] = inct
            return (jnp.max(incr), jnp.max(inct))
        lax.fori_loop(0, NBP // L, a2,
                      (jnp.zeros((), jnp.float32), jnp.zeros((), jnp.float32)))

        def q_of(p):
            p = jnp.minimum(p, nf - 0.5)
            lo = jnp.zeros((L,), jnp.int32)
            hi = jnp.full((L,), K, jnp.int32)
            for _ in range(10):
                mid = (lo + hi) >> 1
                cm = plsc.load_gather(c_t, [mid])
                cond = cm > p
                hi = jnp.where(cond, mid, hi)
                lo = jnp.where(cond, lo, mid + 1)
            l = lo
            lm = jnp.maximum(l - 1, 0)
            ctm1 = plsc.load_gather(c_t, [lm])
            ctm1 = jnp.where(l == 0, 0.0, ctm1)
            cl = plsc.load_gather(c_t, [l])
            hl = jnp.maximum(cl - ctm1, 1.0)
            v = (l.astype(jnp.float32) - 1.0) * INVK + INVK * (p - ctm1) / hl
            return jnp.where(l == 0, 0.0, v)

        def bphase(j, carry):
            sl = pl.ds(j * L, L)
            a = q_of(cx_r[sl])
            vtop = q_of(c_r[sl])
            binv = j * L + lane
            d = jnp.where(binv == 0, 0.0, vtop - a)
            abuf[sl] = a
            dbuf[sl] = d
            return carry
        lax.fori_loop(0, NBP // L, bphase, 0)

        def chunk_c(cidx, acc):
            off = cidx * CH
            pltpu.sync_copy(ref2.at[ch, pl.ds(off, CH)], b0)
            pltpu.sync_copy(sm2.at[b, pl.ds(off, CH)], b1)
            pltpu.sync_copy(src2.at[ch, pl.ds(off, CH)], b2)

            def inner(i, acc2):
                sl = pl.ds(i * L, L)
                rv = b0[sl]
                mv = b1[sl]
                sv = b2[sl]
                r = jnp.minimum(jnp.maximum(rv * 0.5 + 0.5, 0.0), 1.0) * mv
                kf, ki = to_bin(r)
                frac = kf - ki.astype(jnp.float32) + 1.0
                a = plsc.load_gather(abuf, [ki])
                dv = plsc.load_gather(dbuf, [ki])
                matched = a + dv * frac
                s = jnp.minimum(jnp.maximum(sv * 0.5 + 0.5, 0.0), 1.0) * mv
                diff = s - mv * matched
                return acc2 + diff * diff
            return lax.fori_loop(0, CH // L, inner, acc)
        acc = lax.fori_loop(0, nchunk, chunk_c, jnp.zeros((L,), jnp.float32))
        b3[pl.ds(0, L)] = acc
        pltpu.sync_copy(b3.at[pl.ds(0, L)], out.at[pl.ds(ch * L, L)])


def _pallas_loss(ref2, tgt2, sm2, tm2, src2, interpret=False):
    mesh = plsc.VectorSubcoreMesh(core_axis_name="c", subcore_axis_name="s",
                                  num_cores=NCORES, num_subcores=NSUB)
    return pl.kernel(
        _body,
        out_type=jax.ShapeDtypeStruct((NCH * L,), jnp.float32),
        mesh=mesh,
        compiler_params=pltpu.CompilerParams(needs_layout_passes=False),
        scratch_types=[
            pltpu.VMEM((NBP * L,), jnp.float32),
            pltpu.VMEM((NBP * L,), jnp.float32),
            pltpu.VMEM((NBP,), jnp.float32),
            pltpu.VMEM((NBP,), jnp.float32),
            pltpu.VMEM((NBP,), jnp.float32),
            pltpu.VMEM((NBP,), jnp.float32),
            pltpu.VMEM((NBP,), jnp.float32),
            pltpu.VMEM((CH,), jnp.float32),
            pltpu.VMEM((CH,), jnp.float32),
            pltpu.VMEM((CH,), jnp.float32),
            pltpu.VMEM((CH,), jnp.float32),
        ],
        interpret=interpret,
    )(ref2, tgt2, sm2, tm2, src2)


def kernel(src_img, target_img, src_mask, target_mask, ref_img):
    B, C, h, w = src_img.shape
    n = h * w
    src2 = src_img.reshape(B * C, n)
    tgt2 = target_img.reshape(B * C, n)
    ref2 = ref_img.reshape(B * C, n)
    sm2 = src_mask.reshape(B, n)
    tm2 = target_mask.reshape(B, n)
    out = _pallas_loss(ref2, tgt2, sm2, tm2, src2)
    return jnp.sum(out) / (B * C * n)


# speedup vs baseline: 90.5592x; 1.6197x over previous
import jax
import jax.numpy as jnp
from jax import lax
from jax.experimental import pallas as pl
from jax.experimental.pallas import tpu as pltpu
from jax.experimental.pallas import tpu_sc as plsc

K = 512
NB = K + 1
L = 16
NBP = ((NB + L - 1) // L) * L + L
NCORES = 2
NSUB = 16
NCH = 12
CH = 8192
UNROLL = 4
K_F = float(K)
INVK = 1.0 / K


def _prep_body(ref_b, sm_b, tgt_b, tm_b, src_b, ir_b, it_b, kf_b, s_b):
    m = sm_b[...]
    r = jnp.minimum(jnp.maximum(ref_b[...] * 0.5 + 0.5, 0.0), 1.0) * m
    kf = r * K_F
    ir_b[...] = jnp.ceil(kf) * 16.0
    kf_b[...] = kf
    mt = tm_b[...]
    t = jnp.minimum(jnp.maximum(tgt_b[...] * 0.5 + 0.5, 0.0), 1.0) * mt
    it_b[...] = jnp.ceil(t * K_F) * 16.0
    s_b[...] = jnp.minimum(jnp.maximum(src_b[...] * 0.5 + 0.5, 0.0), 1.0) * m


def _prep(ref3, sm3, tgt3, tm3, src3):
    nch, rows, cols = ref3.shape
    rblk = 64
    blk = (1, rblk, cols)
    img_spec = pl.BlockSpec(blk, lambda i, j: (i, j, 0))
    msk_spec = pl.BlockSpec(blk, lambda i, j: (i // 3, j, 0))
    otype = jax.ShapeDtypeStruct(ref3.shape, jnp.float32)
    return pl.pallas_call(
        _prep_body,
        grid=(nch, rows // rblk),
        in_specs=[img_spec, msk_spec, img_spec, msk_spec, img_spec],
        out_specs=[img_spec, img_spec, img_spec, img_spec],
        out_shape=[otype, otype, otype, otype],
    )(ref3, sm3, tgt3, tm3, src3)


def _body(ir2, it2, kf2, s2, sm2, out,
          hist_r, hist_t, cx_r, c_r, c_t, abuf, dbuf, b0, b1, b2,
          sem0, sem1):
    npix = ir2.shape[1]
    nchunk = npix // CH
    nf = jnp.float32(npix)
    core = lax.axis_index("c")
    sub = lax.axis_index("s")
    ch = core * (NCH // NCORES) + sub
    b = ch // 3
    lane = lax.iota(jnp.int32, L)
    onesv = jnp.ones((L,), jnp.float32)
    sems = (sem0, sem1)

    def start_a(cidx, slot):
        off = cidx * CH
        pltpu.make_async_copy(ir2.at[ch, pl.ds(off, CH)], b0.at[slot], sems[slot]).start()
        pltpu.make_async_copy(it2.at[ch, pl.ds(off, CH)], b1.at[slot], sems[slot]).start()

    def wait_a(slot):
        pltpu.make_async_copy(ir2.at[ch, pl.ds(0, CH)], b0.at[slot], sems[slot]).wait()
        pltpu.make_async_copy(it2.at[ch, pl.ds(0, CH)], b1.at[slot], sems[slot]).wait()

    def compute_a(slot):
        def inner(i, carry):
            for u in range(UNROLL):
                sl = pl.ds((i * UNROLL + u) * L, L)
                idxr = b0[slot, sl].astype(jnp.int32) + lane
                idxt = b1[slot, sl].astype(jnp.int32) + lane
                plsc.addupdate_scatter(hist_r, [idxr], onesv)
                plsc.addupdate_scatter(hist_t, [idxt], onesv)
            return carry
        lax.fori_loop(0, CH // L // UNROLL, inner, 0)

    def start_c(cidx, slot):
        off = cidx * CH
        pltpu.make_async_copy(kf2.at[ch, pl.ds(off, CH)], b0.at[slot], sems[slot]).start()
        pltpu.make_async_copy(s2.at[ch, pl.ds(off, CH)], b1.at[slot], sems[slot]).start()
        pltpu.make_async_copy(sm2.at[b, pl.ds(off, CH)], b2.at[slot], sems[slot]).start()

    def wait_c(slot):
        pltpu.make_async_copy(kf2.at[ch, pl.ds(0, CH)], b0.at[slot], sems[slot]).wait()
        pltpu.make_async_copy(s2.at[ch, pl.ds(0, CH)], b1.at[slot], sems[slot]).wait()
        pltpu.make_async_copy(sm2.at[b, pl.ds(0, CH)], b2.at[slot], sems[slot]).wait()

    def compute_c(slot, acc):
        def inner(i, acc2):
            for u in range(UNROLL):
                sl = pl.ds((i * UNROLL + u) * L, L)
                kf = b0[slot, sl]
                sv = b1[slot, sl]
                mv = b2[slot, sl]
                ki = kf.astype(jnp.int32)
                kif = ki.astype(jnp.float32)
                up = kf > kif
                ki = jnp.where(up, ki + 1, ki)
                frac = (kf - kif) + jnp.where(up, 0.0, 1.0)
                a = plsc.load_gather(abuf, [ki])
                dv = plsc.load_gather(dbuf, [ki])
                matched = a + dv * frac
                diff = sv - mv * matched
                acc2 = acc2 + diff * diff
            return acc2
        return lax.fori_loop(0, CH // L // UNROLL, inner, acc)

    @pl.when(sub < (NCH // NCORES))
    def _():
        def zero_body(i, carry):
            z = jnp.zeros((L,), jnp.float32)
            hist_r[pl.ds(i * L, L)] = z
            hist_t[pl.ds(i * L, L)] = z
            return carry
        lax.fori_loop(0, NBP, zero_body, 0)

        start_a(0, 0)

        def pair_a(h, carry):
            c0 = h * 2
            start_a(c0 + 1, 1)
            wait_a(0)
            compute_a(0)

            @pl.when(c0 + 2 < nchunk)
            def _():
                start_a(c0 + 2, 0)
            wait_a(1)
            compute_a(1)
            return carry
        lax.fori_loop(0, nchunk // 2, pair_a, 0)

        def a2(j, carries):
            car_r, car_t = carries
            binv16 = (j * L + lane) * L
            accr = jnp.zeros((L,), jnp.float32)
            acct = jnp.zeros((L,), jnp.float32)
            for l in range(L):
                accr = accr + plsc.load_gather(hist_r, [binv16 + l])
                acct = acct + plsc.load_gather(hist_t, [binv16 + l])
            incr = plsc.cumsum(accr) + car_r
            inct = plsc.cumsum(acct) + car_t
            sl = pl.ds(j * L, L)
            cx_r[sl] = incr - accr
            c_r[sl] = incr
            c_t[sl] = inct
            return (jnp.max(incr), jnp.max(inct))
        lax.fori_loop(0, NBP // L, a2,
                      (jnp.zeros((), jnp.float32), jnp.zeros((), jnp.float32)))

        def q_of(p):
            p = jnp.minimum(p, nf - 0.5)
            lo = jnp.zeros((L,), jnp.int32)
            hi = jnp.full((L,), K, jnp.int32)
            for _ in range(10):
                mid = (lo + hi) >> 1
                cm = plsc.load_gather(c_t, [mid])
                cond = cm > p
                hi = jnp.where(cond, mid, hi)
                lo = jnp.where(cond, lo, mid + 1)
            l = lo
            lm = jnp.maximum(l - 1, 0)
            ctm1 = plsc.load_gather(c_t, [lm])
            ctm1 = jnp.where(l == 0, 0.0, ctm1)
            cl = plsc.load_gather(c_t, [l])
            hl = jnp.maximum(cl - ctm1, 1.0)
            v = (l.astype(jnp.float32) - 1.0) * INVK + INVK * (p - ctm1) / hl
            return jnp.where(l == 0, 0.0, v)

        def bphase(j, carry):
            sl = pl.ds(j * L, L)
            a = q_of(cx_r[sl])
            vtop = q_of(c_r[sl])
            binv = j * L + lane
            d = jnp.where(binv == 0, 0.0, vtop - a)
            abuf[sl] = a
            dbuf[sl] = d
            return carry
        lax.fori_loop(0, NBP // L, bphase, 0)

        start_c(0, 0)

        def pair_c(h, acc):
            c0 = h * 2
            start_c(c0 + 1, 1)
            wait_c(0)
            acc = compute_c(0, acc)

            @pl.when(c0 + 2 < nchunk)
            def _():
                start_c(c0 + 2, 0)
            wait_c(1)
            acc = compute_c(1, acc)
            return acc
        acc = lax.fori_loop(0, nchunk // 2, pair_c, jnp.zeros((L,), jnp.float32))
        b2[0, pl.ds(0, L)] = acc
        pltpu.sync_copy(b2.at[0, pl.ds(0, L)], out.at[pl.ds(ch * L, L)])


def _pallas_loss(ir2, it2, kf2, s2, sm2):
    mesh = plsc.VectorSubcoreMesh(core_axis_name="c", subcore_axis_name="s",
                                  num_cores=NCORES, num_subcores=NSUB)
    return pl.kernel(
        _body,
        out_type=jax.ShapeDtypeStruct((NCH * L,), jnp.float32),
        mesh=mesh,
        compiler_params=pltpu.CompilerParams(needs_layout_passes=False),
        scratch_types=[
            pltpu.VMEM((NBP * L,), jnp.float32),
            pltpu.VMEM((NBP * L,), jnp.float32),
            pltpu.VMEM((NBP,), jnp.float32),
            pltpu.VMEM((NBP,), jnp.float32),
            pltpu.VMEM((NBP,), jnp.float32),
            pltpu.VMEM((NBP,), jnp.float32),
            pltpu.VMEM((NBP,), jnp.float32),
            pltpu.VMEM((2, CH), jnp.float32),
            pltpu.VMEM((2, CH), jnp.float32),
            pltpu.VMEM((2, CH), jnp.float32),
            pltpu.SemaphoreType.DMA,
            pltpu.SemaphoreType.DMA,
        ],
    )(ir2, it2, kf2, s2, sm2)


def kernel(src_img, target_img, src_mask, target_mask, ref_img):
    B, C, h, w = src_img.shape
    n = h * w
    rows = n // 1024
    src3 = src_img.reshape(B * C, rows, 1024)
    tgt3 = target_img.reshape(B * C, rows, 1024)
    ref3 = ref_img.reshape(B * C, rows, 1024)
    sm3 = src_mask.reshape(B, rows, 1024)
    tm3 = target_mask.reshape(B, rows, 1024)
    ir3, it3, kf3, s3 = _prep(ref3, sm3, tgt3, tm3, src3)
    out = _pallas_loss(ir3.reshape(B * C, n), it3.reshape(B * C, n),
                       kf3.reshape(B * C, n), s3.reshape(B * C, n),
                       src_mask.reshape(B, n))
    return jnp.sum(out) / (B * C * n)


# speedup vs baseline: 132.9449x; 1.4680x over previous
import jax
import jax.numpy as jnp
from jax import lax
from jax.experimental import pallas as pl
from jax.experimental.pallas import tpu as pltpu
from jax.experimental.pallas import tpu_sc as plsc

K = 512
NB = K + 1
L = 16
NBP = ((NB + L - 1) // L) * L + L
NCORES = 2
NSUB = 16
NCH = 12
W = 512
CROWS = 16
CH = CROWS * W
UNROLL = 4
K_F = float(K)
INVK = 1.0 / K


def _prep_body(ref_b, sm_b, tgt_b, tm_b, src_b, ir_b, it_b, kf_b, s_b, mv_b):
    m = sm_b[0, 0]
    r = jnp.minimum(jnp.maximum(ref_b[0, 0] * 0.5 + 0.5, 0.0), 1.0) * m
    kf = r * K_F
    ir_b[0] = jnp.ceil(kf) * 16.0
    kf_b[0] = kf
    mt = tm_b[0, 0]
    t = jnp.minimum(jnp.maximum(tgt_b[0, 0] * 0.5 + 0.5, 0.0), 1.0) * mt
    it_b[0] = jnp.ceil(t * K_F) * 16.0
    s_b[0] = jnp.minimum(jnp.maximum(src_b[0, 0] * 0.5 + 0.5, 0.0), 1.0) * m
    mv_b[0] = m


def _prep(ref4, sm4, tgt4, tm4, src4):
    b, c, h, w = ref4.shape
    rblk = 64
    iblk = (1, 1, rblk, w)
    oblk = (1, rblk, w)
    img_spec = pl.BlockSpec(iblk, lambda i, j: (i // 3, i % 3, j, 0))
    msk_spec = pl.BlockSpec(iblk, lambda i, j: (i // 3, 0, j, 0))
    out_spec = pl.BlockSpec(oblk, lambda i, j: (i, j, 0))
    otype = jax.ShapeDtypeStruct((b * c, h, w), jnp.float32)
    return pl.pallas_call(
        _prep_body,
        grid=(b * c, h // rblk),
        in_specs=[img_spec, msk_spec, img_spec, msk_spec, img_spec],
        out_specs=[out_spec] * 5,
        out_shape=[otype] * 5,
    )(ref4, sm4, tgt4, tm4, src4)


def _body(ir2, it2, kf2, s2, mv2, out,
          hist_r, hist_t, cx_r, c_r, c_t, abuf, dbuf, b0, b1, b2,
          sem0, sem1):
    rows_per_ch = ir2.shape[0] // NCH
    npix = rows_per_ch * W
    nchunk = rows_per_ch // CROWS
    nf = jnp.float32(npix)
    core = lax.axis_index("c")
    sub = lax.axis_index("s")
    ch = core * (NCH // NCORES) + sub
    chrow = ch * rows_per_ch
    lane = lax.iota(jnp.int32, L)
    onesv = jnp.ones((L,), jnp.float32)
    sems = (sem0, sem1)

    def start_a(cidx, slot):
        rb = chrow + cidx * CROWS
        pltpu.make_async_copy(ir2.at[pl.ds(rb, CROWS)], b0.at[slot], sems[slot]).start()
        pltpu.make_async_copy(it2.at[pl.ds(rb, CROWS)], b1.at[slot], sems[slot]).start()

    def wait_a(slot):
        pltpu.make_async_copy(ir2.at[pl.ds(0, CROWS)], b0.at[slot], sems[slot]).wait()
        pltpu.make_async_copy(it2.at[pl.ds(0, CROWS)], b1.at[slot], sems[slot]).wait()

    def compute_a(slot):
        def inner(i, carry):
            for u in range(UNROLL):
                v = i * UNROLL + u
                r = v >> 5
                cofs = (v & 31) * L
                idxr = b0[slot, r, pl.ds(cofs, L)].astype(jnp.int32) + lane
                idxt = b1[slot, r, pl.ds(cofs, L)].astype(jnp.int32) + lane
                plsc.addupdate_scatter(hist_r, [idxr], onesv)
                plsc.addupdate_scatter(hist_t, [idxt], onesv)
            return carry
        lax.fori_loop(0, CH // L // UNROLL, inner, 0)

    def start_c(cidx, slot):
        rb = chrow + cidx * CROWS
        pltpu.make_async_copy(kf2.at[pl.ds(rb, CROWS)], b0.at[slot], sems[slot]).start()
        pltpu.make_async_copy(s2.at[pl.ds(rb, CROWS)], b1.at[slot], sems[slot]).start()
        pltpu.make_async_copy(mv2.at[pl.ds(rb, CROWS)], b2.at[slot], sems[slot]).start()

    def wait_c(slot):
        pltpu.make_async_copy(kf2.at[pl.ds(0, CROWS)], b0.at[slot], sems[slot]).wait()
        pltpu.make_async_copy(s2.at[pl.ds(0, CROWS)], b1.at[slot], sems[slot]).wait()
        pltpu.make_async_copy(mv2.at[pl.ds(0, CROWS)], b2.at[slot], sems[slot]).wait()

    def compute_c(slot, acc):
        def inner(i, acc2):
            for u in range(UNROLL):
                v = i * UNROLL + u
                r = v >> 5
                cofs = (v & 31) * L
                kf = b0[slot, r, pl.ds(cofs, L)]
                sv = b1[slot, r, pl.ds(cofs, L)]
                mv = b2[slot, r, pl.ds(cofs, L)]
                ki = kf.astype(jnp.int32)
                kif = ki.astype(jnp.float32)
                up = kf > kif
                ki = jnp.where(up, ki + 1, ki)
                frac = (kf - kif) + jnp.where(up, 0.0, 1.0)
                a = plsc.load_gather(abuf, [ki])
                dv = plsc.load_gather(dbuf, [ki])
                matched = a + dv * frac
                diff = sv - mv * matched
                acc2 = acc2 + diff * diff
            return acc2
        return lax.fori_loop(0, CH // L // UNROLL, inner, acc)

    @pl.when(sub < (NCH // NCORES))
    def _():
        def zero_body(i, carry):
            z = jnp.zeros((L,), jnp.float32)
            hist_r[pl.ds(i * L, L)] = z
            hist_t[pl.ds(i * L, L)] = z
            return carry
        lax.fori_loop(0, NBP, zero_body, 0)

        start_a(0, 0)

        def pair_a(h, carry):
            c0 = h * 2
            start_a(c0 + 1, 1)
            wait_a(0)
            compute_a(0)

            @pl.when(c0 + 2 < nchunk)
            def _():
                start_a(c0 + 2, 0)
            wait_a(1)
            compute_a(1)
            return carry
        lax.fori_loop(0, nchunk // 2, pair_a, 0)

        def a2(j, carries):
            car_r, car_t = carries
            binv16 = (j * L + lane) * L
            accr = jnp.zeros((L,), jnp.float32)
            acct = jnp.zeros((L,), jnp.float32)
            for l in range(L):
                accr = accr + plsc.load_gather(hist_r, [binv16 + l])
                acct = acct + plsc.load_gather(hist_t, [binv16 + l])
            incr = plsc.cumsum(accr) + car_r
            inct = plsc.cumsum(acct) + car_t
            sl = pl.ds(j * L, L)
            cx_r[sl] = incr - accr
            c_r[sl] = incr
            c_t[sl] = inct
            return (jnp.max(incr), jnp.max(inct))
        lax.fori_loop(0, NBP // L, a2,
                      (jnp.zeros((), jnp.float32), jnp.zeros((), jnp.float32)))

        def q_of(p):
            p = jnp.minimum(p, nf - 0.5)
            lo = jnp.zeros((L,), jnp.int32)
            hi = jnp.full((L,), K, jnp.int32)
            for _ in range(10):
                mid = (lo + hi) >> 1
                cm = plsc.load_gather(c_t, [mid])
                cond = cm > p
                hi = jnp.where(cond, mid, hi)
                lo = jnp.where(cond, lo, mid + 1)
            l = lo
            lm = jnp.maximum(l - 1, 0)
            ctm1 = plsc.load_gather(c_t, [lm])
            ctm1 = jnp.where(l == 0, 0.0, ctm1)
            cl = plsc.load_gather(c_t, [l])
            hl = jnp.maximum(cl - ctm1, 1.0)
            v = (l.astype(jnp.float32) - 1.0) * INVK + INVK * (p - ctm1) / hl
            return jnp.where(l == 0, 0.0, v)

        def bphase(j, carry):
            sl = pl.ds(j * L, L)
            a = q_of(cx_r[sl])
            vtop = q_of(c_r[sl])
            binv = j * L + lane
            d = jnp.where(binv == 0, 0.0, vtop - a)
            abuf[sl] = a
            dbuf[sl] = d
            return carry
        lax.fori_loop(0, NBP // L, bphase, 0)

        start_c(0, 0)

        def pair_c(h, acc):
            c0 = h * 2
            start_c(c0 + 1, 1)
            wait_c(0)
            acc = compute_c(0, acc)

            @pl.when(c0 + 2 < nchunk)
            def _():
                start_c(c0 + 2, 0)
            wait_c(1)
            acc = compute_c(1, acc)
            return acc
        acc = lax.fori_loop(0, nchunk // 2, pair_c, jnp.zeros((L,), jnp.float32))
        b2[0, 0, pl.ds(0, L)] = acc
        pltpu.sync_copy(b2.at[0, 0, pl.ds(0, L)], out.at[pl.ds(ch * L, L)])


def _pallas_loss(ir2, it2, kf2, s2, mv2):
    mesh = plsc.VectorSubcoreMesh(core_axis_name="c", subcore_axis_name="s",
                                  num_cores=NCORES, num_subcores=NSUB)
    return pl.kernel(
        _body,
        out_type=jax.ShapeDtypeStruct((NCH * L,), jnp.float32),
        mesh=mesh,
        compiler_params=pltpu.CompilerParams(needs_layout_passes=False),
        scratch_types=[
            pltpu.VMEM((NBP * L,), jnp.float32),
            pltpu.VMEM((NBP * L,), jnp.float32),
            pltpu.VMEM((NBP,), jnp.float32),
            pltpu.VMEM((NBP,), jnp.float32),
            pltpu.VMEM((NBP,), jnp.float32),
            pltpu.VMEM((NBP,), jnp.float32),
            pltpu.VMEM((NBP,), jnp.float32),
            pltpu.VMEM((2, CROWS, W), jnp.float32),
            pltpu.VMEM((2, CROWS, W), jnp.float32),
            pltpu.VMEM((2, CROWS, W), jnp.float32),
            pltpu.SemaphoreType.DMA,
            pltpu.SemaphoreType.DMA,
        ],
    )(ir2, it2, kf2, s2, mv2)


def kernel(src_img, target_img, src_mask, target_mask, ref_img):
    B, C, h, w = src_img.shape
    n = h * w
    ir3, it3, kf3, s3, mv3 = _prep(ref_img, src_mask, target_img,
                                   target_mask, src_img)
    out = _pallas_loss(ir3.reshape(B * C * h, w), it3.reshape(B * C * h, w),
                       kf3.reshape(B * C * h, w), s3.reshape(B * C * h, w),
                       mv3.reshape(B * C * h, w))
    return jnp.sum(out) / (B * C * n)


# speedup vs baseline: 137.4962x; 1.0342x over previous
import jax
import jax.numpy as jnp
from jax import lax
from jax.experimental import pallas as pl
from jax.experimental.pallas import tpu as pltpu
from jax.experimental.pallas import tpu_sc as plsc

K = 512
NB = K + 1
L = 16
NBP = ((NB + L - 1) // L) * L + L
NCORES = 2
NSUB = 16
NCH = 12
W = 512
CROWS = 16
CH = CROWS * W
UNROLL = 8
K_F = float(K)
INVK = 1.0 / K


def _prep_body(ref_b, sm_b, tgt_b, tm_b, src_b, ir_b, it_b, kf_b, s_b, mv_b):
    m = sm_b[0, 0]
    r = jnp.minimum(jnp.maximum(ref_b[0, 0] * 0.5 + 0.5, 0.0), 1.0) * m
    kf = r * K_F
    ir_b[0] = jnp.ceil(kf) * 16.0
    kf_b[0] = kf
    mt = tm_b[0, 0]
    t = jnp.minimum(jnp.maximum(tgt_b[0, 0] * 0.5 + 0.5, 0.0), 1.0) * mt
    it_b[0] = jnp.ceil(t * K_F) * 16.0
    s_b[0] = jnp.minimum(jnp.maximum(src_b[0, 0] * 0.5 + 0.5, 0.0), 1.0) * m
    mv_b[0] = m


def _prep(ref4, sm4, tgt4, tm4, src4):
    b, c, h, w = ref4.shape
    rblk = 64
    iblk = (1, 1, rblk, w)
    oblk = (1, rblk, w)
    img_spec = pl.BlockSpec(iblk, lambda i, j: (i // 3, i % 3, j, 0))
    msk_spec = pl.BlockSpec(iblk, lambda i, j: (i // 3, 0, j, 0))
    out_spec = pl.BlockSpec(oblk, lambda i, j: (i, j, 0))
    otype = jax.ShapeDtypeStruct((b * c, h, w), jnp.float32)
    return pl.pallas_call(
        _prep_body,
        grid=(b * c, h // rblk),
        in_specs=[img_spec, msk_spec, img_spec, msk_spec, img_spec],
        out_specs=[out_spec] * 5,
        out_shape=[otype] * 5,
    )(ref4, sm4, tgt4, tm4, src4)


def _body(ir2, it2, kf2, s2, mv2, out,
          hist_r, hist_t, cx_r, c_r, c_t, abuf, dbuf, b0, b1, b2,
          sem0, sem1):
    rows_per_ch = ir2.shape[0] // NCH
    npix = rows_per_ch * W
    nchunk = rows_per_ch // CROWS
    nf = jnp.float32(npix)
    core = lax.axis_index("c")
    sub = lax.axis_index("s")
    ch = core * (NCH // NCORES) + sub
    chrow = ch * rows_per_ch
    lane = lax.iota(jnp.int32, L)
    onesv = jnp.ones((L,), jnp.float32)
    sems = (sem0, sem1)

    def start_a(cidx, slot):
        rb = chrow + cidx * CROWS
        pltpu.make_async_copy(ir2.at[pl.ds(rb, CROWS)], b0.at[slot], sems[slot]).start()
        pltpu.make_async_copy(it2.at[pl.ds(rb, CROWS)], b1.at[slot], sems[slot]).start()

    def wait_a(slot):
        pltpu.make_async_copy(ir2.at[pl.ds(0, CROWS)], b0.at[slot], sems[slot]).wait()
        pltpu.make_async_copy(it2.at[pl.ds(0, CROWS)], b1.at[slot], sems[slot]).wait()

    def compute_a(slot):
        def inner(i, carry):
            for u in range(UNROLL):
                v = i * UNROLL + u
                r = v >> 5
                cofs = (v & 31) * L
                idxr = b0[slot, r, pl.ds(cofs, L)].astype(jnp.int32) + lane
                idxt = b1[slot, r, pl.ds(cofs, L)].astype(jnp.int32) + lane
                plsc.addupdate_scatter(hist_r, [idxr], onesv)
                plsc.addupdate_scatter(hist_t, [idxt], onesv)
            return carry
        lax.fori_loop(0, CH // L // UNROLL, inner, 0)

    def start_c(cidx, slot):
        rb = chrow + cidx * CROWS
        pltpu.make_async_copy(kf2.at[pl.ds(rb, CROWS)], b0.at[slot], sems[slot]).start()
        pltpu.make_async_copy(s2.at[pl.ds(rb, CROWS)], b1.at[slot], sems[slot]).start()
        pltpu.make_async_copy(mv2.at[pl.ds(rb, CROWS)], b2.at[slot], sems[slot]).start()

    def wait_c(slot):
        pltpu.make_async_copy(kf2.at[pl.ds(0, CROWS)], b0.at[slot], sems[slot]).wait()
        pltpu.make_async_copy(s2.at[pl.ds(0, CROWS)], b1.at[slot], sems[slot]).wait()
        pltpu.make_async_copy(mv2.at[pl.ds(0, CROWS)], b2.at[slot], sems[slot]).wait()

    def compute_c(slot, acc):
        def inner(i, acc2):
            for u in range(UNROLL):
                v = i * UNROLL + u
                r = v >> 5
                cofs = (v & 31) * L
                kf = b0[slot, r, pl.ds(cofs, L)]
                sv = b1[slot, r, pl.ds(cofs, L)]
                mv = b2[slot, r, pl.ds(cofs, L)]
                ki = kf.astype(jnp.int32)
                kif = ki.astype(jnp.float32)
                up = kf > kif
                ki = jnp.where(up, ki + 1, ki)
                frac = (kf - kif) + jnp.where(up, 0.0, 1.0)
                a = plsc.load_gather(abuf, [ki])
                dv = plsc.load_gather(dbuf, [ki])
                matched = a + dv * frac
                diff = sv - mv * matched
                acc2 = acc2 + diff * diff
            return acc2
        return lax.fori_loop(0, CH // L // UNROLL, inner, acc)

    @pl.when(sub < (NCH // NCORES))
    def _():
        def zero_body(i, carry):
            z = jnp.zeros((L,), jnp.float32)
            hist_r[pl.ds(i * L, L)] = z
            hist_t[pl.ds(i * L, L)] = z
            return carry
        lax.fori_loop(0, NBP, zero_body, 0)

        start_a(0, 0)

        def pair_a(h, carry):
            c0 = h * 2
            start_a(c0 + 1, 1)
            wait_a(0)
            compute_a(0)

            @pl.when(c0 + 2 < nchunk)
            def _():
                start_a(c0 + 2, 0)
            wait_a(1)
            compute_a(1)
            return carry
        lax.fori_loop(0, nchunk // 2, pair_a, 0)

        def a2(j, carries):
            car_r, car_t = carries
            binv16 = (j * L + lane) * L
            accr = jnp.zeros((L,), jnp.float32)
            acct = jnp.zeros((L,), jnp.float32)
            for l in range(L):
                accr = accr + plsc.load_gather(hist_r, [binv16 + l])
                acct = acct + plsc.load_gather(hist_t, [binv16 + l])
            incr = plsc.cumsum(accr) + car_r
            inct = plsc.cumsum(acct) + car_t
            sl = pl.ds(j * L, L)
            cx_r[sl] = incr - accr
            c_r[sl] = incr
            c_t[sl] = inct
            return (jnp.max(incr), jnp.max(inct))
        lax.fori_loop(0, NBP // L, a2,
                      (jnp.zeros((), jnp.float32), jnp.zeros((), jnp.float32)))

        def q_of(p):
            p = jnp.minimum(p, nf - 0.5)
            lo = jnp.zeros((L,), jnp.int32)
            hi = jnp.full((L,), K, jnp.int32)
            for _ in range(10):
                mid = (lo + hi) >> 1
                cm = plsc.load_gather(c_t, [mid])
                cond = cm > p
                hi = jnp.where(cond, mid, hi)
                lo = jnp.where(cond, lo, mid + 1)
            l = lo
            lm = jnp.maximum(l - 1, 0)
            ctm1 = plsc.load_gather(c_t, [lm])
            ctm1 = jnp.where(l == 0, 0.0, ctm1)
            cl = plsc.load_gather(c_t, [l])
            hl = jnp.maximum(cl - ctm1, 1.0)
            v = (l.astype(jnp.float32) - 1.0) * INVK + INVK * (p - ctm1) / hl
            return jnp.where(l == 0, 0.0, v)

        def bphase(j, carry):
            sl = pl.ds(j * L, L)
            a = q_of(cx_r[sl])
            vtop = q_of(c_r[sl])
            binv = j * L + lane
            d = jnp.where(binv == 0, 0.0, vtop - a)
            abuf[sl] = a
            dbuf[sl] = d
            return carry
        lax.fori_loop(0, NBP // L, bphase, 0)

        start_c(0, 0)

        def pair_c(h, acc):
            c0 = h * 2
            start_c(c0 + 1, 1)
            wait_c(0)
            acc = compute_c(0, acc)

            @pl.when(c0 + 2 < nchunk)
            def _():
                start_c(c0 + 2, 0)
            wait_c(1)
            acc = compute_c(1, acc)
            return acc
        acc = lax.fori_loop(0, nchunk // 2, pair_c, jnp.zeros((L,), jnp.float32))
        b2[0, 0, pl.ds(0, L)] = acc
        pltpu.sync_copy(b2.at[0, 0, pl.ds(0, L)], out.at[pl.ds(ch * L, L)])


def _pallas_loss(ir2, it2, kf2, s2, mv2):
    mesh = plsc.VectorSubcoreMesh(core_axis_name="c", subcore_axis_name="s",
                                  num_cores=NCORES, num_subcores=NSUB)
    return pl.kernel(
        _body,
        out_type=jax.ShapeDtypeStruct((NCH * L,), jnp.float32),
        mesh=mesh,
        compiler_params=pltpu.CompilerParams(needs_layout_passes=False),
        scratch_types=[
            pltpu.VMEM((NBP * L,), jnp.float32),
            pltpu.VMEM((NBP * L,), jnp.float32),
            pltpu.VMEM((NBP,), jnp.float32),
            pltpu.VMEM((NBP,), jnp.float32),
            pltpu.VMEM((NBP,), jnp.float32),
            pltpu.VMEM((NBP,), jnp.float32),
            pltpu.VMEM((NBP,), jnp.float32),
            pltpu.VMEM((2, CROWS, W), jnp.float32),
            pltpu.VMEM((2, CROWS, W), jnp.float32),
            pltpu.VMEM((2, CROWS, W), jnp.float32),
            pltpu.SemaphoreType.DMA,
            pltpu.SemaphoreType.DMA,
        ],
    )(ir2, it2, kf2, s2, mv2)


def kernel(src_img, target_img, src_mask, target_mask, ref_img):
    B, C, h, w = src_img.shape
    n = h * w
    ir3, it3, kf3, s3, mv3 = _prep(ref_img, src_mask, target_img,
                                   target_mask, src_img)
    out = _pallas_loss(ir3.reshape(B * C * h, w), it3.reshape(B * C * h, w),
                       kf3.reshape(B * C * h, w), s3.reshape(B * C * h, w),
                       mv3.reshape(B * C * h, w))
    return jnp.sum(out) / (B * C * n)


# speedup vs baseline: 156.7941x; 1.1404x over previous
import jax
import jax.numpy as jnp
from jax import lax
from jax.experimental import pallas as pl
from jax.experimental.pallas import tpu as pltpu
from jax.experimental.pallas import tpu_sc as plsc

K = 512
NB = K + 1
L = 16
NBP = ((NB + L - 1) // L) * L + L
NCORES = 2
NSUB = 16
NCH = 12
W = 512
CROWS = 16
CH = CROWS * W
UNROLL = 8
K_F = float(K)
INVK = 1.0 / K


def _prep_body(ref_b, sm_b, tgt_b, tm_b, src_b, key_b, kf_b, s_b, mv_b):
    m = sm_b[0, 0]
    r = jnp.minimum(jnp.maximum(ref_b[0, 0] * 0.5 + 0.5, 0.0), 1.0) * m
    kf = r * K_F
    kf_b[0] = kf
    mt = tm_b[0, 0]
    t = jnp.minimum(jnp.maximum(tgt_b[0, 0] * 0.5 + 0.5, 0.0), 1.0) * mt
    ir16 = (jnp.ceil(kf) * 16.0).astype(jnp.int32)
    it16 = (jnp.ceil(t * K_F) * 16.0).astype(jnp.int32)
    key_b[0] = ir16 | (it16 << 16)
    s_b[0] = jnp.minimum(jnp.maximum(src_b[0, 0] * 0.5 + 0.5, 0.0), 1.0) * m
    mv_b[0] = m


def _prep(ref4, sm4, tgt4, tm4, src4):
    b, c, h, w = ref4.shape
    rblk = 64
    iblk = (1, 1, rblk, w)
    oblk = (1, rblk, w)
    img_spec = pl.BlockSpec(iblk, lambda i, j: (i // 3, i % 3, j, 0))
    msk_spec = pl.BlockSpec(iblk, lambda i, j: (i // 3, 0, j, 0))
    out_spec = pl.BlockSpec(oblk, lambda i, j: (i, j, 0))
    otype = jax.ShapeDtypeStruct((b * c, h, w), jnp.float32)
    ktype = jax.ShapeDtypeStruct((b * c, h, w), jnp.int32)
    return pl.pallas_call(
        _prep_body,
        grid=(b * c, h // rblk),
        in_specs=[img_spec, msk_spec, img_spec, msk_spec, img_spec],
        out_specs=[out_spec] * 4,
        out_shape=[ktype, otype, otype, otype],
    )(ref4, sm4, tgt4, tm4, src4)


def _body(key2, kf2, s2, mv2, out,
          hist_r, hist_t, cx_r, c_r, c_t, abuf, dbuf, bk, b0, b1, b2,
          sem0, sem1):
    rows_per_ch = key2.shape[0] // NCH
    npix = rows_per_ch * W
    nchunk = rows_per_ch // CROWS
    nf = jnp.float32(npix)
    core = lax.axis_index("c")
    sub = lax.axis_index("s")
    ch = core * (NCH // NCORES) + sub
    chrow = ch * rows_per_ch
    lane = lax.iota(jnp.int32, L)
    onesv = jnp.ones((L,), jnp.float32)
    sems = (sem0, sem1)

    def start_a(cidx, slot):
        rb = chrow + cidx * CROWS
        pltpu.make_async_copy(key2.at[pl.ds(rb, CROWS)], bk.at[slot], sems[slot]).start()

    def wait_a(slot):
        pltpu.make_async_copy(key2.at[pl.ds(0, CROWS)], bk.at[slot], sems[slot]).wait()

    def compute_a(slot):
        def inner(i, carry):
            for u in range(UNROLL):
                v = i * UNROLL + u
                r = v >> 5
                cofs = (v & 31) * L
                w = bk[slot, r, pl.ds(cofs, L)]
                idxr = (w & 0xFFFF) + lane
                idxt = lax.shift_right_logical(w, 16) + lane
                plsc.addupdate_scatter(hist_r, [idxr], onesv)
                plsc.addupdate_scatter(hist_t, [idxt], onesv)
            return carry
        lax.fori_loop(0, CH // L // UNROLL, inner, 0)

    def start_c(cidx, slot):
        rb = chrow + cidx * CROWS
        pltpu.make_async_copy(kf2.at[pl.ds(rb, CROWS)], b0.at[slot], sems[slot]).start()
        pltpu.make_async_copy(s2.at[pl.ds(rb, CROWS)], b1.at[slot], sems[slot]).start()
        pltpu.make_async_copy(mv2.at[pl.ds(rb, CROWS)], b2.at[slot], sems[slot]).start()

    def wait_c(slot):
        pltpu.make_async_copy(kf2.at[pl.ds(0, CROWS)], b0.at[slot], sems[slot]).wait()
        pltpu.make_async_copy(s2.at[pl.ds(0, CROWS)], b1.at[slot], sems[slot]).wait()
        pltpu.make_async_copy(mv2.at[pl.ds(0, CROWS)], b2.at[slot], sems[slot]).wait()

    def compute_c(slot, acc):
        def inner(i, acc2):
            for u in range(UNROLL):
                v = i * UNROLL + u
                r = v >> 5
                cofs = (v & 31) * L
                kf = b0[slot, r, pl.ds(cofs, L)]
                sv = b1[slot, r, pl.ds(cofs, L)]
                mv = b2[slot, r, pl.ds(cofs, L)]
                ki = kf.astype(jnp.int32)
                kif = ki.astype(jnp.float32)
                up = kf > kif
                ki = jnp.where(up, ki + 1, ki)
                frac = (kf - kif) + jnp.where(up, 0.0, 1.0)
                a = plsc.load_gather(abuf, [ki])
                dv = plsc.load_gather(dbuf, [ki])
                matched = a + dv * frac
                diff = sv - mv * matched
                acc2 = acc2 + diff * diff
            return acc2
        return lax.fori_loop(0, CH // L // UNROLL, inner, acc)

    @pl.when(sub < (NCH // NCORES))
    def _():
        def zero_body(i, carry):
            z = jnp.zeros((L,), jnp.float32)
            hist_r[pl.ds(i * L, L)] = z
            hist_t[pl.ds(i * L, L)] = z
            return carry
        lax.fori_loop(0, NBP, zero_body, 0)

        start_a(0, 0)

        def pair_a(h, carry):
            c0 = h * 2
            start_a(c0 + 1, 1)
            wait_a(0)
            compute_a(0)

            @pl.when(c0 + 2 < nchunk)
            def _():
                start_a(c0 + 2, 0)
            wait_a(1)
            compute_a(1)
            return carry
        lax.fori_loop(0, nchunk // 2, pair_a, 0)

        def a2(j, carries):
            car_r, car_t = carries
            binv16 = (j * L + lane) * L
            accr = jnp.zeros((L,), jnp.float32)
            acct = jnp.zeros((L,), jnp.float32)
            for l in range(L):
                accr = accr + plsc.load_gather(hist_r, [binv16 + l])
                acct = acct + plsc.load_gather(hist_t, [binv16 + l])
            incr = plsc.cumsum(accr) + car_r
            inct = plsc.cumsum(acct) + car_t
            sl = pl.ds(j * L, L)
            cx_r[sl] = incr - accr
            c_r[sl] = incr
            c_t[sl] = inct
            return (jnp.max(incr), jnp.max(inct))
        lax.fori_loop(0, NBP // L, a2,
                      (jnp.zeros((), jnp.float32), jnp.zeros((), jnp.float32)))

        def q_of(p):
            p = jnp.minimum(p, nf - 0.5)
            lo = jnp.zeros((L,), jnp.int32)
            hi = jnp.full((L,), K, jnp.int32)
            for _ in range(10):
                mid = (lo + hi) >> 1
                cm = plsc.load_gather(c_t, [mid])
                cond = cm > p
                hi = jnp.where(cond, mid, hi)
                lo = jnp.where(cond, lo, mid + 1)
            l = lo
            lm = jnp.maximum(l - 1, 0)
            ctm1 = plsc.load_gather(c_t, [lm])
            ctm1 = jnp.where(l == 0, 0.0, ctm1)
            cl = plsc.load_gather(c_t, [l])
            hl = jnp.maximum(cl - ctm1, 1.0)
            v = (l.astype(jnp.float32) - 1.0) * INVK + INVK * (p - ctm1) / hl
            return jnp.where(l == 0, 0.0, v)

        def bphase(j, carry):
            sl = pl.ds(j * L, L)
            a = q_of(cx_r[sl])
            vtop = q_of(c_r[sl])
            binv = j * L + lane
            d = jnp.where(binv == 0, 0.0, vtop - a)
            abuf[sl] = a
            dbuf[sl] = d
            return carry
        lax.fori_loop(0, NBP // L, bphase, 0)

        start_c(0, 0)

        def pair_c(h, acc):
            c0 = h * 2
            start_c(c0 + 1, 1)
            wait_c(0)
            acc = compute_c(0, acc)

            @pl.when(c0 + 2 < nchunk)
            def _():
                start_c(c0 + 2, 0)
            wait_c(1)
            acc = compute_c(1, acc)
            return acc
        acc = lax.fori_loop(0, nchunk // 2, pair_c, jnp.zeros((L,), jnp.float32))
        b2[0, 0, pl.ds(0, L)] = acc
        pltpu.sync_copy(b2.at[0, 0, pl.ds(0, L)], out.at[pl.ds(ch * L, L)])


def _pallas_loss(key2, kf2, s2, mv2):
    mesh = plsc.VectorSubcoreMesh(core_axis_name="c", subcore_axis_name="s",
                                  num_cores=NCORES, num_subcores=NSUB)
    return pl.kernel(
        _body,
        out_type=jax.ShapeDtypeStruct((NCH * L,), jnp.float32),
        mesh=mesh,
        compiler_params=pltpu.CompilerParams(needs_layout_passes=False),
        scratch_types=[
            pltpu.VMEM((NBP * L,), jnp.float32),
            pltpu.VMEM((NBP * L,), jnp.float32),
            pltpu.VMEM((NBP,), jnp.float32),
            pltpu.VMEM((NBP,), jnp.float32),
            pltpu.VMEM((NBP,), jnp.float32),
            pltpu.VMEM((NBP,), jnp.float32),
            pltpu.VMEM((NBP,), jnp.float32),
            pltpu.VMEM((2, CROWS, W), jnp.int32),
            pltpu.VMEM((2, CROWS, W), jnp.float32),
            pltpu.VMEM((2, CROWS, W), jnp.float32),
            pltpu.VMEM((2, CROWS, W), jnp.float32),
            pltpu.SemaphoreType.DMA,
            pltpu.SemaphoreType.DMA,
        ],
    )(key2, kf2, s2, mv2)


def kernel(src_img, target_img, src_mask, target_mask, ref_img):
    B, C, h, w = src_img.shape
    n = h * w
    key3, kf3, s3, mv3 = _prep(ref_img, src_mask, target_img,
                               target_mask, src_img)
    out = _pallas_loss(key3.reshape(B * C * h, w), kf3.reshape(B * C * h, w),
                       s3.reshape(B * C * h, w), mv3.reshape(B * C * h, w))
    return jnp.sum(out) / (B * C * n)


# speedup vs baseline: 191.4604x; 1.2211x over previous
import jax
import jax.numpy as jnp
from jax import lax
from jax.experimental import pallas as pl
from jax.experimental.pallas import tpu as pltpu
from jax.experimental.pallas import tpu_sc as plsc

K = 512
L = 16
NCORES = 2
NSUB = 16
NCH = 12
CPC = NCH // NCORES
SLICE = 48
NBP = NSUB * SLICE
NV = SLICE // L
W = 512
TROWS = 32
CROWS = 16
CH = CROWS * W
UNROLL = 8
K_F = float(K)
INVK = 1.0 / K


def _prep_body(ref_b, sm_b, tgt_b, tm_b, src_b, key_b, kf_b, s_b, mv_b):
    m = sm_b[0, 0]
    r = jnp.minimum(jnp.maximum(ref_b[0, 0] * 0.5 + 0.5, 0.0), 1.0) * m
    kf = r * K_F
    kf_b[0] = kf
    mt = tm_b[0, 0]
    t = jnp.minimum(jnp.maximum(tgt_b[0, 0] * 0.5 + 0.5, 0.0), 1.0) * mt
    ir16 = (jnp.ceil(kf) * 16.0).astype(jnp.int32)
    it16 = (jnp.ceil(t * K_F) * 16.0).astype(jnp.int32)
    key_b[0] = ir16 | (it16 << 16)
    s_b[0] = jnp.minimum(jnp.maximum(src_b[0, 0] * 0.5 + 0.5, 0.0), 1.0) * m
    mv_b[0] = m


def _prep(ref4, sm4, tgt4, tm4, src4):
    b, c, h, w = ref4.shape
    rblk = 64
    iblk = (1, 1, rblk, w)
    oblk = (1, rblk, w)
    img_spec = pl.BlockSpec(iblk, lambda i, j: (i // 3, i % 3, j, 0))
    msk_spec = pl.BlockSpec(iblk, lambda i, j: (i // 3, 0, j, 0))
    out_spec = pl.BlockSpec(oblk, lambda i, j: (i, j, 0))
    otype = jax.ShapeDtypeStruct((b * c, h, w), jnp.float32)
    ktype = jax.ShapeDtypeStruct((b * c, h, w), jnp.int32)
    return pl.pallas_call(
        _prep_body,
        grid=(b * c, h // rblk),
        in_specs=[img_spec, msk_spec, img_spec, msk_spec, img_spec],
        out_specs=[out_spec] * 4,
        out_shape=[ktype, otype, otype, otype],
    )(ref4, sm4, tgt4, tm4, src4)


def _body(key2, kf2, s2, mv2, out,
          hist_r, hist_t, adbuf, ctfull, tbuf, sbuf, cbuf,
          bk, b0, b1, b2,
          sh_tot, sh_st, sh_ct, sh_ad,
          sem0, sem1):
    rows_per_ch = key2.shape[0] // NCH
    npix = rows_per_ch * W
    nf = jnp.float32(npix)
    core = lax.axis_index("c")
    sub = lax.axis_index("s")
    lane = lax.iota(jnp.int32, L)
    onesv = jnp.ones((L,), jnp.float32)
    zidx = jnp.zeros((L,), jnp.int32)
    sems = (sem0, sem1)

    def start_a(rb, slot):
        pltpu.make_async_copy(key2.at[pl.ds(rb, CROWS)], bk.at[slot], sems[slot]).start()

    def wait_a(slot):
        pltpu.make_async_copy(key2.at[pl.ds(0, CROWS)], bk.at[slot], sems[slot]).wait()

    def compute_a(slot):
        def inner(i, carry):
            for u in range(UNROLL):
                v = i * UNROLL + u
                r = v >> 5
                cofs = (v & 31) * L
                w = bk[slot, r, pl.ds(cofs, L)]
                idxr = (w & 0xFFFF) + lane
                idxt = lax.shift_right_logical(w, 16) + lane
                plsc.addupdate_scatter(hist_r, [idxr], onesv)
                plsc.addupdate_scatter(hist_t, [idxt], onesv)
            return carry
        lax.fori_loop(0, CH // L // UNROLL, inner, 0)

    def start_c(rb, slot):
        pltpu.make_async_copy(kf2.at[pl.ds(rb, CROWS)], b0.at[slot], sems[slot]).start()
        pltpu.make_async_copy(s2.at[pl.ds(rb, CROWS)], b1.at[slot], sems[slot]).start()
        pltpu.make_async_copy(mv2.at[pl.ds(rb, CROWS)], b2.at[slot], sems[slot]).start()

    def wait_c(slot):
        pltpu.make_async_copy(kf2.at[pl.ds(0, CROWS)], b0.at[slot], sems[slot]).wait()
        pltpu.make_async_copy(s2.at[pl.ds(0, CROWS)], b1.at[slot], sems[slot]).wait()
        pltpu.make_async_copy(mv2.at[pl.ds(0, CROWS)], b2.at[slot], sems[slot]).wait()

    def compute_c(slot, acc):
        def inner(i, acc2):
            for u in range(UNROLL):
                v = i * UNROLL + u
                r = v >> 5
                cofs = (v & 31) * L
                kf = b0[slot, r, pl.ds(cofs, L)]
                sv = b1[slot, r, pl.ds(cofs, L)]
                mv = b2[slot, r, pl.ds(cofs, L)]
                ki = kf.astype(jnp.int32)
                kif = ki.astype(jnp.float32)
                up = kf > kif
                ki = jnp.where(up, ki + 1, ki)
                frac = (kf - kif) + jnp.where(up, 0.0, 1.0)
                a = plsc.load_gather(adbuf, [zidx, ki])
                dv = plsc.load_gather(adbuf, [zidx + 1, ki])
                matched = a + dv * frac
                diff = sv - mv * matched
                acc2 = acc2 + diff * diff
            return acc2
        return lax.fori_loop(0, CH // L // UNROLL, inner, acc)

    def q_of(p):
        p = jnp.minimum(p, nf - 0.5)
        lo = jnp.zeros((L,), jnp.int32)
        hi = jnp.full((L,), K, jnp.int32)
        for _ in range(10):
            mid = (lo + hi) >> 1
            cm = plsc.load_gather(ctfull, [mid])
            cond = cm > p
            hi = jnp.where(cond, mid, hi)
            lo = jnp.where(cond, lo, mid + 1)
        l = lo
        lm = jnp.maximum(l - 1, 0)
        ctm1 = plsc.load_gather(ctfull, [lm])
        ctm1 = jnp.where(l == 0, 0.0, ctm1)
        cl = plsc.load_gather(ctfull, [l])
        hl = jnp.maximum(cl - ctm1, 1.0)
        v = (l.astype(jnp.float32) - 1.0) * INVK + INVK * (p - ctm1) / hl
        return jnp.where(l == 0, 0.0, v)

    def zero_body(i, carry):
        z = jnp.zeros((L,), jnp.float32)
        hist_r[pl.ds(i * L, L)] = z
        hist_t[pl.ds(i * L, L)] = z
        return carry
    lax.fori_loop(0, NBP, zero_body, 0)

    def chan(ci, acc):
        chrow = (core * CPC + ci) * rows_per_ch
        myrow = chrow + sub * TROWS

        start_a(myrow, 0)
        start_a(myrow + CROWS, 1)
        wait_a(0)
        compute_a(0)
        wait_a(1)
        compute_a(1)

        def a2(j, carry):
            base = (j * L + lane) * L

            def gsum(hist, b):
                acc2 = jnp.zeros((L,), jnp.float32)
                for l in range(L):
                    acc2 = acc2 + plsc.load_gather(hist, [b + l])
                return acc2
            accr = gsum(hist_r, base)
            acct = gsum(hist_t, base)
            s_id = j // NV
            pos = (j % NV) * L
            tbuf[s_id, pl.ds(pos, L)] = accr
            tbuf[s_id, pl.ds(64 + pos, L)] = acct
            z = jnp.zeros((L,), jnp.float32)
            for l2 in range(L):
                hist_r[pl.ds((j * L + l2) * L, L)] = z
                hist_t[pl.ds((j * L + l2) * L, L)] = z
            return carry
        lax.fori_loop(0, NBP // L, a2, 0)
        pltpu.sync_copy(tbuf, sh_tot.at[sub])
        plsc.subcore_barrier()

        pltpu.sync_copy(sh_tot.at[:, sub], sbuf)
        cnt_r, cnt_t = [], []
        for v in range(NV):
            ar = jnp.zeros((L,), jnp.float32)
            at_ = jnp.zeros((L,), jnp.float32)
            for t in range(NSUB):
                ar = ar + sbuf[t, pl.ds(v * L, L)]
                at_ = at_ + sbuf[t, pl.ds(64 + v * L, L)]
            cnt_r.append(ar)
            cnt_t.append(at_)
        inc_r, inc_t = [], []
        car = jnp.zeros((), jnp.float32)
        for v in range(NV):
            inc = plsc.cumsum(cnt_r[v]) + car
            inc_r.append(inc)
            car = jnp.max(inc)
        tot_r = car
        car = jnp.zeros((), jnp.float32)
        for v in range(NV):
            inc = plsc.cumsum(cnt_t[v]) + car
            inc_t.append(inc)
            car = jnp.max(inc)
        tot_t = car
        stv = jnp.where(lane == 0, tot_r, jnp.where(lane == 1, tot_t, 0.0))
        cbuf[pl.ds(0, L)] = stv
        pltpu.sync_copy(cbuf, sh_st.at[sub])
        plsc.subcore_barrier()

        pltpu.sync_copy(sh_st, sbuf)
        totr_all = plsc.load_gather(sbuf, [lane, zidx])
        tott_all = plsc.load_gather(sbuf, [lane, zidx + 1])
        before = lane < sub
        pref_r = jnp.sum(jnp.where(before, totr_all, 0.0))
        pref_t = jnp.sum(jnp.where(before, tott_all, 0.0))
        cr_g = [inc_r[v] + pref_r for v in range(NV)]
        cx_g = [cr_g[v] - cnt_r[v] for v in range(NV)]
        for v in range(NV):
            cbuf[pl.ds(v * L, L)] = inc_t[v] + pref_t
        pltpu.sync_copy(cbuf, sh_ct.at[sub])
        plsc.subcore_barrier()
        pltpu.sync_copy(sh_ct, sbuf)
        for j in range(NBP // L):
            ctfull[pl.ds(j * L, L)] = sbuf[j // NV, pl.ds((j % NV) * L, L)]

        for v in range(NV):
            a = q_of(cx_g[v])
            vtop = q_of(cr_g[v])
            gbin = sub * SLICE + v * L + lane
            d = jnp.where(gbin == 0, 0.0, vtop - a)
            cbuf[pl.ds(v * L, L)] = a
            cbuf[pl.ds(64 + v * L, L)] = d
        pltpu.sync_copy(cbuf, sh_ad.at[sub])
        plsc.subcore_barrier()
        pltpu.sync_copy(sh_ad, sbuf)
        for j in range(NBP // L):
            adbuf[0, pl.ds(j * L, L)] = sbuf[j // NV, pl.ds((j % NV) * L, L)]
            adbuf[1, pl.ds(j * L, L)] = sbuf[j // NV, pl.ds(64 + (j % NV) * L, L)]

        start_c(myrow, 0)
        start_c(myrow + CROWS, 1)
        wait_c(0)
        acc = compute_c(0, acc)
        wait_c(1)
        acc = compute_c(1, acc)
        return acc

    acc = lax.fori_loop(0, CPC, chan, jnp.zeros((L,), jnp.float32))
    wid = core * NSUB + sub
    b2[0, 0, pl.ds(0, L)] = acc
    pltpu.sync_copy(b2.at[0, 0, pl.ds(0, L)], out.at[pl.ds(wid * L, L)])


def _pallas_loss(key2, kf2, s2, mv2):
    mesh = plsc.VectorSubcoreMesh(core_axis_name="c", subcore_axis_name="s",
                                  num_cores=NCORES, num_subcores=NSUB)
    return pl.kernel(
        _body,
        out_type=jax.ShapeDtypeStruct((NCORES * NSUB * L,), jnp.float32),
        mesh=mesh,
        compiler_params=pltpu.CompilerParams(needs_layout_passes=False),
        scratch_types=[
            pltpu.VMEM((NBP * L,), jnp.float32),
            pltpu.VMEM((NBP * L,), jnp.float32),
            pltpu.VMEM((2, NBP), jnp.float32),
            pltpu.VMEM((NBP,), jnp.float32),
            pltpu.VMEM((NSUB, 128), jnp.float32),
            pltpu.VMEM((NSUB, 128), jnp.float32),
            pltpu.VMEM((128,), jnp.float32),
            pltpu.VMEM((2, CROWS, W), jnp.int32),
            pltpu.VMEM((2, CROWS, W), jnp.float32),
            pltpu.VMEM((2, CROWS, W), jnp.float32),
            pltpu.VMEM((2, CROWS, W), jnp.float32),
            pltpu.VMEM_SHARED((NSUB, NSUB, 128), jnp.float32),
            pltpu.VMEM_SHARED((NSUB, 128), jnp.float32),
            pltpu.VMEM_SHARED((NSUB, 128), jnp.float32),
            pltpu.VMEM_SHARED((NSUB, 128), jnp.float32),
            pltpu.SemaphoreType.DMA,
            pltpu.SemaphoreType.DMA,
        ],
    )(key2, kf2, s2, mv2)


def kernel(src_img, target_img, src_mask, target_mask, ref_img):
    B, C, h, w = src_img.shape
    n = h * w
    key3, kf3, s3, mv3 = _prep(ref_img, src_mask, target_img,
                               target_mask, src_img)
    out = _pallas_loss(key3.reshape(B * C * h, w), kf3.reshape(B * C * h, w),
                       s3.reshape(B * C * h, w), mv3.reshape(B * C * h, w))
    return jnp.sum(out) / (B * C * n)


# speedup vs baseline: 223.3150x; 1.1664x over previous
import jax
import jax.numpy as jnp
from jax import lax
from jax.experimental import pallas as pl
from jax.experimental.pallas import tpu as pltpu
from jax.experimental.pallas import tpu_sc as plsc

K = 512
L = 16
NCORES = 2
NSUB = 16
NCH = 12
CPC = NCH // NCORES
SLICE = 48
NBP = NSUB * SLICE
NV = SLICE // L
W = 512
TROWS = 32
CROWS = 16
CH = CROWS * W
UNROLL = 8
K_F = float(K)
INVK = 1.0 / K


def _prep_body(ref_b, sm_b, tgt_b, tm_b, src_b, key_b, kf_b, s_b, mv_b):
    m = sm_b[0, 0]
    r = jnp.minimum(jnp.maximum(ref_b[0, 0] * 0.5 + 0.5, 0.0), 1.0) * m
    kf = r * K_F
    kf_b[0] = kf
    mt = tm_b[0, 0]
    t = jnp.minimum(jnp.maximum(tgt_b[0, 0] * 0.5 + 0.5, 0.0), 1.0) * mt
    ir16 = (jnp.ceil(kf) * 16.0).astype(jnp.int32)
    it16 = (jnp.ceil(t * K_F) * 16.0).astype(jnp.int32)
    key_b[0] = ir16 | (it16 << 16)
    s_b[0] = jnp.minimum(jnp.maximum(src_b[0, 0] * 0.5 + 0.5, 0.0), 1.0) * m
    mv_b[0] = m


def _prep(ref4, sm4, tgt4, tm4, src4):
    b, c, h, w = ref4.shape
    rblk = 64
    iblk = (1, 1, rblk, w)
    oblk = (1, rblk, w)
    img_spec = pl.BlockSpec(iblk, lambda i, j: (i // 3, i % 3, j, 0))
    msk_spec = pl.BlockSpec(iblk, lambda i, j: (i // 3, 0, j, 0))
    out_spec = pl.BlockSpec(oblk, lambda i, j: (i, j, 0))
    otype = jax.ShapeDtypeStruct((b * c, h, w), jnp.float32)
    ktype = jax.ShapeDtypeStruct((b * c, h, w), jnp.int32)
    return pl.pallas_call(
        _prep_body,
        grid=(b * c, h // rblk),
        in_specs=[img_spec, msk_spec, img_spec, msk_spec, img_spec],
        out_specs=[out_spec] * 4,
        out_shape=[ktype, otype, otype, otype],
    )(ref4, sm4, tgt4, tm4, src4)


def _body(key2, kf2, s2, mv2, out,
          hist_r, hist_t, adbuf, ctfull, tbuf, sbuf, cbuf,
          bk, b0, b1, b2,
          sh_tot, sh_st, sh_ct, sh_ad,
          sem0, sem1):
    rows_per_ch = key2.shape[0] // NCH
    npix = rows_per_ch * W
    nf = jnp.float32(npix)
    core = lax.axis_index("c")
    sub = lax.axis_index("s")
    lane = lax.iota(jnp.int32, L)
    onesv = jnp.ones((L,), jnp.float32)
    zidx = jnp.zeros((L,), jnp.int32)
    sems = (sem0, sem1)

    def start_a(rb, slot):
        pltpu.make_async_copy(key2.at[pl.ds(rb, CROWS)], bk.at[slot], sems[slot]).start()

    def wait_a(slot):
        pltpu.make_async_copy(key2.at[pl.ds(0, CROWS)], bk.at[slot], sems[slot]).wait()

    def compute_a(slot):
        @plsc.parallel_loop(0, CH // L, unroll=UNROLL)
        def _(v):
            r = v >> 5
            cofs = (v & 31) * L
            w = bk[slot, r, pl.ds(cofs, L)]
            idxr = (w & 0xFFFF) + lane
            idxt = lax.shift_right_logical(w, 16) + lane
            plsc.addupdate_scatter(hist_r, [idxr], onesv)
            plsc.addupdate_scatter(hist_t, [idxt], onesv)

    def start_c(rb, slot):
        pltpu.make_async_copy(kf2.at[pl.ds(rb, CROWS)], b0.at[slot], sems[slot]).start()
        pltpu.make_async_copy(s2.at[pl.ds(rb, CROWS)], b1.at[slot], sems[slot]).start()
        pltpu.make_async_copy(mv2.at[pl.ds(rb, CROWS)], b2.at[slot], sems[slot]).start()

    def wait_c(slot):
        pltpu.make_async_copy(kf2.at[pl.ds(0, CROWS)], b0.at[slot], sems[slot]).wait()
        pltpu.make_async_copy(s2.at[pl.ds(0, CROWS)], b1.at[slot], sems[slot]).wait()
        pltpu.make_async_copy(mv2.at[pl.ds(0, CROWS)], b2.at[slot], sems[slot]).wait()

    def compute_c(slot, acc):
        @plsc.parallel_loop(0, CH // L, unroll=UNROLL, carry=acc)
        def inner(v, acc2):
            r = v >> 5
            cofs = (v & 31) * L
            kf = b0[slot, r, pl.ds(cofs, L)]
            sv = b1[slot, r, pl.ds(cofs, L)]
            mv = b2[slot, r, pl.ds(cofs, L)]
            ki = kf.astype(jnp.int32)
            kif = ki.astype(jnp.float32)
            up = kf > kif
            ki = jnp.where(up, ki + 1, ki)
            frac = (kf - kif) + jnp.where(up, 0.0, 1.0)
            a = plsc.load_gather(adbuf, [zidx, ki])
            dv = plsc.load_gather(adbuf, [zidx + 1, ki])
            matched = a + dv * frac
            diff = sv - mv * matched
            return acc2 + diff * diff
        return inner

    def q_of(p):
        p = jnp.minimum(p, nf - 0.5)
        lo = jnp.zeros((L,), jnp.int32)
        hi = jnp.full((L,), K, jnp.int32)
        for _ in range(10):
            mid = (lo + hi) >> 1
            cm = plsc.load_gather(ctfull, [mid])
            cond = cm > p
            hi = jnp.where(cond, mid, hi)
            lo = jnp.where(cond, lo, mid + 1)
        l = lo
        lm = jnp.maximum(l - 1, 0)
        ctm1 = plsc.load_gather(ctfull, [lm])
        ctm1 = jnp.where(l == 0, 0.0, ctm1)
        cl = plsc.load_gather(ctfull, [l])
        hl = jnp.maximum(cl - ctm1, 1.0)
        v = (l.astype(jnp.float32) - 1.0) * INVK + INVK * (p - ctm1) / hl
        return jnp.where(l == 0, 0.0, v)

    def zero_body(i, carry):
        z = jnp.zeros((L,), jnp.float32)
        hist_r[pl.ds(i * L, L)] = z
        hist_t[pl.ds(i * L, L)] = z
        return carry
    lax.fori_loop(0, NBP, zero_body, 0)

    def chan(ci, acc):
        chrow = (core * CPC + ci) * rows_per_ch
        myrow = chrow + sub * TROWS

        start_a(myrow, 0)
        start_a(myrow + CROWS, 1)
        wait_a(0)
        compute_a(0)
        wait_a(1)
        compute_a(1)

        def a2(j, carry):
            base = (j * L + lane) * L

            def gsum(hist, b):
                acc2 = jnp.zeros((L,), jnp.float32)
                for l in range(L):
                    acc2 = acc2 + plsc.load_gather(hist, [b + l])
                return acc2
            accr = gsum(hist_r, base)
            acct = gsum(hist_t, base)
            s_id = j // NV
            pos = (j % NV) * L
            tbuf[s_id, pl.ds(pos, L)] = accr
            tbuf[s_id, pl.ds(64 + pos, L)] = acct
            z = jnp.zeros((L,), jnp.float32)
            for l2 in range(L):
                hist_r[pl.ds((j * L + l2) * L, L)] = z
                hist_t[pl.ds((j * L + l2) * L, L)] = z
            return carry
        lax.fori_loop(0, NBP // L, a2, 0)
        pltpu.sync_copy(tbuf, sh_tot.at[sub])
        plsc.subcore_barrier()

        pltpu.sync_copy(sh_tot.at[:, sub], sbuf)
        cnt_r, cnt_t = [], []
        for v in range(NV):
            ar = jnp.zeros((L,), jnp.float32)
            at_ = jnp.zeros((L,), jnp.float32)
            for t in range(NSUB):
                ar = ar + sbuf[t, pl.ds(v * L, L)]
                at_ = at_ + sbuf[t, pl.ds(64 + v * L, L)]
            cnt_r.append(ar)
            cnt_t.append(at_)
        inc_r, inc_t = [], []
        car = jnp.zeros((), jnp.float32)
        for v in range(NV):
            inc = plsc.cumsum(cnt_r[v]) + car
            inc_r.append(inc)
            car = jnp.max(inc)
        tot_r = car
        car = jnp.zeros((), jnp.float32)
        for v in range(NV):
            inc = plsc.cumsum(cnt_t[v]) + car
            inc_t.append(inc)
            car = jnp.max(inc)
        tot_t = car
        stv = jnp.where(lane == 0, tot_r, jnp.where(lane == 1, tot_t, 0.0))
        cbuf[pl.ds(0, L)] = stv
        pltpu.sync_copy(cbuf, sh_st.at[sub])
        plsc.subcore_barrier()

        pltpu.sync_copy(sh_st, sbuf)
        totr_all = plsc.load_gather(sbuf, [lane, zidx])
        tott_all = plsc.load_gather(sbuf, [lane, zidx + 1])
        before = lane < sub
        pref_r = jnp.sum(jnp.where(before, totr_all, 0.0))
        pref_t = jnp.sum(jnp.where(before, tott_all, 0.0))
        cr_g = [inc_r[v] + pref_r for v in range(NV)]
        cx_g = [cr_g[v] - cnt_r[v] for v in range(NV)]
        for v in range(NV):
            cbuf[pl.ds(v * L, L)] = inc_t[v] + pref_t
        pltpu.sync_copy(cbuf, sh_ct.at[sub])
        plsc.subcore_barrier()
        pltpu.sync_copy(sh_ct, sbuf)
        for j in range(NBP // L):
            ctfull[pl.ds(j * L, L)] = sbuf[j // NV, pl.ds((j % NV) * L, L)]

        for v in range(NV):
            a = q_of(cx_g[v])
            vtop = q_of(cr_g[v])
            gbin = sub * SLICE + v * L + lane
            d = jnp.where(gbin == 0, 0.0, vtop - a)
            cbuf[pl.ds(v * L, L)] = a
            cbuf[pl.ds(64 + v * L, L)] = d
        pltpu.sync_copy(cbuf, sh_ad.at[sub])
        plsc.subcore_barrier()
        pltpu.sync_copy(sh_ad, sbuf)
        for j in range(NBP // L):
            adbuf[0, pl.ds(j * L, L)] = sbuf[j // NV, pl.ds((j % NV) * L, L)]
            adbuf[1, pl.ds(j * L, L)] = sbuf[j // NV, pl.ds(64 + (j % NV) * L, L)]

        start_c(myrow, 0)
        start_c(myrow + CROWS, 1)
        wait_c(0)
        acc = compute_c(0, acc)
        wait_c(1)
        acc = compute_c(1, acc)
        return acc

    acc = lax.fori_loop(0, CPC, chan, jnp.zeros((L,), jnp.float32))
    wid = core * NSUB + sub
    b2[0, 0, pl.ds(0, L)] = acc
    pltpu.sync_copy(b2.at[0, 0, pl.ds(0, L)], out.at[pl.ds(wid * L, L)])


def _pallas_loss(key2, kf2, s2, mv2):
    mesh = plsc.VectorSubcoreMesh(core_axis_name="c", subcore_axis_name="s",
                                  num_cores=NCORES, num_subcores=NSUB)
    return pl.kernel(
        _body,
        out_type=jax.ShapeDtypeStruct((NCORES * NSUB * L,), jnp.float32),
        mesh=mesh,
        compiler_params=pltpu.CompilerParams(needs_layout_passes=False),
        scratch_types=[
            pltpu.VMEM((NBP * L,), jnp.float32),
            pltpu.VMEM((NBP * L,), jnp.float32),
            pltpu.VMEM((2, NBP), jnp.float32),
            pltpu.VMEM((NBP,), jnp.float32),
            pltpu.VMEM((NSUB, 128), jnp.float32),
            pltpu.VMEM((NSUB, 128), jnp.float32),
            pltpu.VMEM((128,), jnp.float32),
            pltpu.VMEM((2, CROWS, W), jnp.int32),
            pltpu.VMEM((2, CROWS, W), jnp.float32),
            pltpu.VMEM((2, CROWS, W), jnp.float32),
            pltpu.VMEM((2, CROWS, W), jnp.float32),
            pltpu.VMEM_SHARED((NSUB, NSUB, 128), jnp.float32),
            pltpu.VMEM_SHARED((NSUB, 128), jnp.float32),
            pltpu.VMEM_SHARED((NSUB, 128), jnp.float32),
            pltpu.VMEM_SHARED((NSUB, 128), jnp.float32),
            pltpu.SemaphoreType.DMA,
            pltpu.SemaphoreType.DMA,
        ],
    )(key2, kf2, s2, mv2)


def kernel(src_img, target_img, src_mask, target_mask, ref_img):
    B, C, h, w = src_img.shape
    n = h * w
    key3, kf3, s3, mv3 = _prep(ref_img, src_mask, target_img,
                               target_mask, src_img)
    out = _pallas_loss(key3.reshape(B * C * h, w), kf3.reshape(B * C * h, w),
                       s3.reshape(B * C * h, w), mv3.reshape(B * C * h, w))
    return jnp.sum(out) / (B * C * n)


# speedup vs baseline: 248.1195x; 1.1111x over previous
import jax
import jax.numpy as jnp
from jax import lax
from jax.experimental import pallas as pl
from jax.experimental.pallas import tpu as pltpu
from jax.experimental.pallas import tpu_sc as plsc

K = 512
L = 16
NCORES = 2
NSUB = 16
NCH = 12
CPC = NCH // NCORES
SLICE = 48
NBP = NSUB * SLICE
NV = SLICE // L
W = 512
TROWS = 32
CROWS = 16
CH = CROWS * W
UNROLL = 8
K_F = float(K)
INVK = 1.0 / K


def _prep_body(ref_b, sm_b, tgt_b, tm_b, src_b, key_b, kf_b, s_b, mv_b):
    m = sm_b[0, 0]
    r = jnp.minimum(jnp.maximum(ref_b[0, 0] * 0.5 + 0.5, 0.0), 1.0) * m
    kf = r * K_F
    kf_b[0] = kf
    mt = tm_b[0, 0]
    t = jnp.minimum(jnp.maximum(tgt_b[0, 0] * 0.5 + 0.5, 0.0), 1.0) * mt
    ir16 = (jnp.ceil(kf) * 16.0).astype(jnp.int32)
    it16 = (jnp.ceil(t * K_F) * 16.0).astype(jnp.int32)
    key_b[0] = ir16 | (it16 << 16)
    s_b[0] = jnp.minimum(jnp.maximum(src_b[0, 0] * 0.5 + 0.5, 0.0), 1.0) * m
    mv_b[0] = m


def _prep(ref4, sm4, tgt4, tm4, src4):
    b, c, h, w = ref4.shape
    rblk = 128
    iblk = (1, 1, rblk, w)
    oblk = (1, rblk, w)
    img_spec = pl.BlockSpec(iblk, lambda i, j: (i // 3, i % 3, j, 0))
    msk_spec = pl.BlockSpec(iblk, lambda i, j: (i // 3, 0, j, 0))
    out_spec = pl.BlockSpec(oblk, lambda i, j: (i, j, 0))
    otype = jax.ShapeDtypeStruct((b * c, h, w), jnp.float32)
    ktype = jax.ShapeDtypeStruct((b * c, h, w), jnp.int32)
    return pl.pallas_call(
        _prep_body,
        grid=(b * c, h // rblk),
        in_specs=[img_spec, msk_spec, img_spec, msk_spec, img_spec],
        out_specs=[out_spec] * 4,
        out_shape=[ktype, otype, otype, otype],
    )(ref4, sm4, tgt4, tm4, src4)


def _body(key2, kf2, s2, mv2, out,
          hist_r, hist_t, adbuf, ctfull, tbuf, sbuf, cbuf,
          bk, b0, b1, b2,
          sh_tot, sh_st, sh_ct, sh_ad,
          sem0, sem1):
    rows_per_ch = key2.shape[0] // NCH
    npix = rows_per_ch * W
    nf = jnp.float32(npix)
    core = lax.axis_index("c")
    sub = lax.axis_index("s")
    lane = lax.iota(jnp.int32, L)
    onesv = jnp.ones((L,), jnp.float32)
    zidx = jnp.zeros((L,), jnp.int32)
    sems = (sem0, sem1)

    def start_a(rb, slot):
        pltpu.make_async_copy(key2.at[pl.ds(rb, CROWS)], bk.at[slot], sems[slot]).start()

    def wait_a(slot):
        pltpu.make_async_copy(key2.at[pl.ds(0, CROWS)], bk.at[slot], sems[slot]).wait()

    def compute_a(slot):
        @plsc.parallel_loop(0, CH // L, unroll=UNROLL)
        def _(v):
            r = v >> 5
            cofs = (v & 31) * L
            w = bk[slot, r, pl.ds(cofs, L)]
            idxr = (w & 0xFFFF) + lane
            idxt = lax.shift_right_logical(w, 16) + lane
            plsc.addupdate_scatter(hist_r, [idxr], onesv)
            plsc.addupdate_scatter(hist_t, [idxt], onesv)

    def start_c(rb, slot):
        pltpu.make_async_copy(kf2.at[pl.ds(rb, CROWS)], b0.at[slot], sems[slot]).start()
        pltpu.make_async_copy(s2.at[pl.ds(rb, CROWS)], b1.at[slot], sems[slot]).start()
        pltpu.make_async_copy(mv2.at[pl.ds(rb, CROWS)], b2.at[slot], sems[slot]).start()

    def wait_c(slot):
        pltpu.make_async_copy(kf2.at[pl.ds(0, CROWS)], b0.at[slot], sems[slot]).wait()
        pltpu.make_async_copy(s2.at[pl.ds(0, CROWS)], b1.at[slot], sems[slot]).wait()
        pltpu.make_async_copy(mv2.at[pl.ds(0, CROWS)], b2.at[slot], sems[slot]).wait()

    def compute_c(slot, acc):
        @plsc.parallel_loop(0, CH // L, unroll=UNROLL, carry=acc)
        def inner(v, acc2):
            r = v >> 5
            cofs = (v & 31) * L
            kf = b0[slot, r, pl.ds(cofs, L)]
            sv = b1[slot, r, pl.ds(cofs, L)]
            mv = b2[slot, r, pl.ds(cofs, L)]
            ki = kf.astype(jnp.int32)
            kif = ki.astype(jnp.float32)
            up = kf > kif
            ki = jnp.where(up, ki + 1, ki)
            frac = (kf - kif) + jnp.where(up, 0.0, 1.0)
            a = plsc.load_gather(adbuf, [zidx, ki])
            dv = plsc.load_gather(adbuf, [zidx + 1, ki])
            matched = a + dv * frac
            diff = sv - mv * matched
            return acc2 + diff * diff
        return inner

    def q_of(p):
        p = jnp.minimum(p, nf - 0.5)
        lo = jnp.zeros((L,), jnp.int32)
        hi = jnp.full((L,), K, jnp.int32)
        for _ in range(10):
            mid = (lo + hi) >> 1
            cm = plsc.load_gather(ctfull, [mid])
            cond = cm > p
            hi = jnp.where(cond, mid, hi)
            lo = jnp.where(cond, lo, mid + 1)
        l = lo
        lm = jnp.maximum(l - 1, 0)
        ctm1 = plsc.load_gather(ctfull, [lm])
        ctm1 = jnp.where(l == 0, 0.0, ctm1)
        cl = plsc.load_gather(ctfull, [l])
        hl = jnp.maximum(cl - ctm1, 1.0)
        v = (l.astype(jnp.float32) - 1.0) * INVK + INVK * (p - ctm1) / hl
        return jnp.where(l == 0, 0.0, v)

    def zero_body(i, carry):
        z = jnp.zeros((L,), jnp.float32)
        hist_r[pl.ds(i * L, L)] = z
        hist_t[pl.ds(i * L, L)] = z
        return carry
    lax.fori_loop(0, NBP, zero_body, 0)

    def chan(ci, acc):
        chrow = (core * CPC + ci) * rows_per_ch
        myrow = chrow + sub * TROWS

        start_a(myrow, 0)
        start_a(myrow + CROWS, 1)
        wait_a(0)
        compute_a(0)
        wait_a(1)
        compute_a(1)

        def a2(j, carry):
            base = (j * L + lane) * L

            def gsum(hist, b):
                acc2 = jnp.zeros((L,), jnp.float32)
                for l in range(L):
                    acc2 = acc2 + plsc.load_gather(hist, [b + l])
                return acc2
            accr = gsum(hist_r, base)
            acct = gsum(hist_t, base)
            s_id = j // NV
            pos = (j % NV) * L
            tbuf[s_id, pl.ds(pos, L)] = accr
            tbuf[s_id, pl.ds(64 + pos, L)] = acct
            z = jnp.zeros((L,), jnp.float32)
            for l2 in range(L):
                hist_r[pl.ds((j * L + l2) * L, L)] = z
                hist_t[pl.ds((j * L + l2) * L, L)] = z
            return carry
        lax.fori_loop(0, NBP // L, a2, 0)
        pltpu.sync_copy(tbuf, sh_tot.at[sub])
        plsc.subcore_barrier()

        pltpu.sync_copy(sh_tot.at[:, sub], sbuf)
        cnt_r, cnt_t = [], []
        for v in range(NV):
            ar = jnp.zeros((L,), jnp.float32)
            at_ = jnp.zeros((L,), jnp.float32)
            for t in range(NSUB):
                ar = ar + sbuf[t, pl.ds(v * L, L)]
                at_ = at_ + sbuf[t, pl.ds(64 + v * L, L)]
            cnt_r.append(ar)
            cnt_t.append(at_)
        inc_r, inc_t = [], []
        car = jnp.zeros((), jnp.float32)
        for v in range(NV):
            inc = plsc.cumsum(cnt_r[v]) + car
            inc_r.append(inc)
            car = jnp.max(inc)
        tot_r = car
        car = jnp.zeros((), jnp.float32)
        for v in range(NV):
            inc = plsc.cumsum(cnt_t[v]) + car
            inc_t.append(inc)
            car = jnp.max(inc)
        tot_t = car
        stv = jnp.where(lane == 0, tot_r, jnp.where(lane == 1, tot_t, 0.0))
        cbuf[pl.ds(0, L)] = stv
        pltpu.sync_copy(cbuf, sh_st.at[sub])
        plsc.subcore_barrier()

        pltpu.sync_copy(sh_st, sbuf)
        totr_all = plsc.load_gather(sbuf, [lane, zidx])
        tott_all = plsc.load_gather(sbuf, [lane, zidx + 1])
        before = lane < sub
        pref_r = jnp.sum(jnp.where(before, totr_all, 0.0))
        pref_t = jnp.sum(jnp.where(before, tott_all, 0.0))
        cr_g = [inc_r[v] + pref_r for v in range(NV)]
        cx_g = [cr_g[v] - cnt_r[v] for v in range(NV)]
        for v in range(NV):
            cbuf[pl.ds(v * L, L)] = inc_t[v] + pref_t
        pltpu.sync_copy(cbuf, sh_ct.at[sub])
        plsc.subcore_barrier()
        pltpu.sync_copy(sh_ct, sbuf)
        for j in range(NBP // L):
            ctfull[pl.ds(j * L, L)] = sbuf[j // NV, pl.ds((j % NV) * L, L)]

        for v in range(NV):
            a = q_of(cx_g[v])
            vtop = q_of(cr_g[v])
            gbin = sub * SLICE + v * L + lane
            d = jnp.where(gbin == 0, 0.0, vtop - a)
            cbuf[pl.ds(v * L, L)] = a
            cbuf[pl.ds(64 + v * L, L)] = d
        pltpu.sync_copy(cbuf, sh_ad.at[sub])
        plsc.subcore_barrier()
        pltpu.sync_copy(sh_ad, sbuf)
        for j in range(NBP // L):
            adbuf[0, pl.ds(j * L, L)] = sbuf[j // NV, pl.ds((j % NV) * L, L)]
            adbuf[1, pl.ds(j * L, L)] = sbuf[j // NV, pl.ds(64 + (j % NV) * L, L)]

        start_c(myrow, 0)
        start_c(myrow + CROWS, 1)
        wait_c(0)
        acc = compute_c(0, acc)
        wait_c(1)
        acc = compute_c(1, acc)
        return acc

    acc = lax.fori_loop(0, CPC, chan, jnp.zeros((L,), jnp.float32))
    wid = core * NSUB + sub
    b2[0, 0, pl.ds(0, L)] = acc
    pltpu.sync_copy(b2.at[0, 0, pl.ds(0, L)], out.at[pl.ds(wid * L, L)])


def _pallas_loss(key2, kf2, s2, mv2):
    mesh = plsc.VectorSubcoreMesh(core_axis_name="c", subcore_axis_name="s",
                                  num_cores=NCORES, num_subcores=NSUB)
    return pl.kernel(
        _body,
        out_type=jax.ShapeDtypeStruct((NCORES * NSUB * L,), jnp.float32),
        mesh=mesh,
        compiler_params=pltpu.CompilerParams(needs_layout_passes=False),
        scratch_types=[
            pltpu.VMEM((NBP * L,), jnp.float32),
            pltpu.VMEM((NBP * L,), jnp.float32),
            pltpu.VMEM((2, NBP), jnp.float32),
            pltpu.VMEM((NBP,), jnp.float32),
            pltpu.VMEM((NSUB, 128), jnp.float32),
            pltpu.VMEM((NSUB, 128), jnp.float32),
            pltpu.VMEM((128,), jnp.float32),
            pltpu.VMEM((2, CROWS, W), jnp.int32),
            pltpu.VMEM((2, CROWS, W), jnp.float32),
            pltpu.VMEM((2, CROWS, W), jnp.float32),
            pltpu.VMEM((2, CROWS, W), jnp.float32),
            pltpu.VMEM_SHARED((NSUB, NSUB, 128), jnp.float32),
            pltpu.VMEM_SHARED((NSUB, 128), jnp.float32),
            pltpu.VMEM_SHARED((NSUB, 128), jnp.float32),
            pltpu.VMEM_SHARED((NSUB, 128), jnp.float32),
            pltpu.SemaphoreType.DMA,
            pltpu.SemaphoreType.DMA,
        ],
    )(key2, kf2, s2, mv2)


def kernel(src_img, target_img, src_mask, target_mask, ref_img):
    B, C, h, w = src_img.shape
    n = h * w
    key3, kf3, s3, mv3 = _prep(ref_img, src_mask, target_img,
                               target_mask, src_img)
    out = _pallas_loss(key3.reshape(B * C * h, w), kf3.reshape(B * C * h, w),
                       s3.reshape(B * C * h, w), mv3.reshape(B * C * h, w))
    return jnp.sum(out) / (B * C * n)


# speedup vs baseline: 264.2360x; 1.0650x over previous
import jax
import jax.numpy as jnp
from jax import lax
from jax.experimental import pallas as pl
from jax.experimental.pallas import tpu as pltpu
from jax.experimental.pallas import tpu_sc as plsc

K = 512
L = 16
NCORES = 2
NSUB = 16
NCH = 12
CPC = NCH // NCORES
SLICE = 48
NBP = NSUB * SLICE
NV = SLICE // L
W = 512
TROWS = 32
CROWS = 16
CH = CROWS * W
UNROLL = 8
K_F = float(K)
INVK = 1.0 / K


def _prep_body(ref_b, sm_b, tgt_b, tm_b, src_b, key_b, kf_b, s_b, mv_b):
    m = sm_b[0, 0]
    r = jnp.minimum(jnp.maximum(ref_b[0, 0] * 0.5 + 0.5, 0.0), 1.0) * m
    kf = r * K_F
    kf_b[0] = kf
    mt = tm_b[0, 0]
    t = jnp.minimum(jnp.maximum(tgt_b[0, 0] * 0.5 + 0.5, 0.0), 1.0) * mt
    ir16 = (jnp.ceil(kf) * 16.0).astype(jnp.int32)
    it16 = (jnp.ceil(t * K_F) * 16.0).astype(jnp.int32)
    key_b[0] = ir16 | (it16 << 16)
    s_b[0] = jnp.minimum(jnp.maximum(src_b[0, 0] * 0.5 + 0.5, 0.0), 1.0) * m
    mv_b[0] = m


def _prep(ref4, sm4, tgt4, tm4, src4):
    b, c, h, w = ref4.shape
    rblk = 256
    iblk = (1, 1, rblk, w)
    oblk = (1, rblk, w)
    img_spec = pl.BlockSpec(iblk, lambda i, j: (i // 3, i % 3, j, 0))
    msk_spec = pl.BlockSpec(iblk, lambda i, j: (i // 3, 0, j, 0))
    out_spec = pl.BlockSpec(oblk, lambda i, j: (i, j, 0))
    otype = jax.ShapeDtypeStruct((b * c, h, w), jnp.float32)
    ktype = jax.ShapeDtypeStruct((b * c, h, w), jnp.int32)
    return pl.pallas_call(
        _prep_body,
        grid=(b * c, h // rblk),
        in_specs=[img_spec, msk_spec, img_spec, msk_spec, img_spec],
        out_specs=[out_spec] * 4,
        out_shape=[ktype, otype, otype, otype],
    )(ref4, sm4, tgt4, tm4, src4)


def _body(key2, kf2, s2, mv2, out,
          hist_r, hist_t, adbuf, ctfull, tbuf, sbuf, cbuf,
          bk, b0, b1, b2,
          sh_tot, sh_st, sh_ct, sh_ad,
          sem0, sem1):
    rows_per_ch = key2.shape[0] // NCH
    npix = rows_per_ch * W
    nf = jnp.float32(npix)
    core = lax.axis_index("c")
    sub = lax.axis_index("s")
    lane = lax.iota(jnp.int32, L)
    onesv = jnp.ones((L,), jnp.float32)
    zidx = jnp.zeros((L,), jnp.int32)
    sems = (sem0, sem1)

    def start_a(rb, slot):
        pltpu.make_async_copy(key2.at[pl.ds(rb, CROWS)], bk.at[slot], sems[slot]).start()

    def wait_a(slot):
        pltpu.make_async_copy(key2.at[pl.ds(0, CROWS)], bk.at[slot], sems[slot]).wait()

    def compute_a(slot):
        @plsc.parallel_loop(0, CH // L, unroll=UNROLL)
        def _(v):
            r = v >> 5
            cofs = (v & 31) * L
            w = bk[slot, r, pl.ds(cofs, L)]
            idxr = (w & 0xFFFF) + lane
            idxt = lax.shift_right_logical(w, 16) + lane
            plsc.addupdate_scatter(hist_r, [idxr], onesv)
            plsc.addupdate_scatter(hist_t, [idxt], onesv)

    def start_c(rb, slot):
        pltpu.make_async_copy(kf2.at[pl.ds(rb, CROWS)], b0.at[slot], sems[slot]).start()
        pltpu.make_async_copy(s2.at[pl.ds(rb, CROWS)], b1.at[slot], sems[slot]).start()
        pltpu.make_async_copy(mv2.at[pl.ds(rb, CROWS)], b2.at[slot], sems[slot]).start()

    def wait_c(slot):
        pltpu.make_async_copy(kf2.at[pl.ds(0, CROWS)], b0.at[slot], sems[slot]).wait()
        pltpu.make_async_copy(s2.at[pl.ds(0, CROWS)], b1.at[slot], sems[slot]).wait()
        pltpu.make_async_copy(mv2.at[pl.ds(0, CROWS)], b2.at[slot], sems[slot]).wait()

    def compute_c(slot, acc):
        @plsc.parallel_loop(0, CH // L, unroll=UNROLL, carry=acc)
        def inner(v, acc2):
            r = v >> 5
            cofs = (v & 31) * L
            kf = b0[slot, r, pl.ds(cofs, L)]
            sv = b1[slot, r, pl.ds(cofs, L)]
            mv = b2[slot, r, pl.ds(cofs, L)]
            ki = kf.astype(jnp.int32)
            kif = ki.astype(jnp.float32)
            up = kf > kif
            ki = jnp.where(up, ki + 1, ki)
            frac = (kf - kif) + jnp.where(up, 0.0, 1.0)
            a = plsc.load_gather(adbuf, [zidx, ki])
            dv = plsc.load_gather(adbuf, [zidx + 1, ki])
            matched = a + dv * frac
            diff = sv - mv * matched
            return acc2 + diff * diff
        return inner

    def q_of(p):
        p = jnp.minimum(p, nf - 0.5)
        lo = jnp.zeros((L,), jnp.int32)
        hi = jnp.full((L,), K, jnp.int32)
        for _ in range(10):
            mid = (lo + hi) >> 1
            cm = plsc.load_gather(ctfull, [mid])
            cond = cm > p
            hi = jnp.where(cond, mid, hi)
            lo = jnp.where(cond, lo, mid + 1)
        l = lo
        lm = jnp.maximum(l - 1, 0)
        ctm1 = plsc.load_gather(ctfull, [lm])
        ctm1 = jnp.where(l == 0, 0.0, ctm1)
        cl = plsc.load_gather(ctfull, [l])
        hl = jnp.maximum(cl - ctm1, 1.0)
        v = (l.astype(jnp.float32) - 1.0) * INVK + INVK * (p - ctm1) / hl
        return jnp.where(l == 0, 0.0, v)

    def zero_body(i, carry):
        z = jnp.zeros((L,), jnp.float32)
        hist_r[pl.ds(i * L, L)] = z
        hist_t[pl.ds(i * L, L)] = z
        return carry
    lax.fori_loop(0, NBP, zero_body, 0)

    def chan(ci, acc):
        chrow = (core * CPC + ci) * rows_per_ch
        myrow = chrow + sub * TROWS

        start_a(myrow, 0)
        start_a(myrow + CROWS, 1)
        wait_a(0)
        compute_a(0)
        wait_a(1)
        compute_a(1)

        def a2(j, carry):
            base = (j * L + lane) * L

            def gsum(hist, b):
                acc2 = jnp.zeros((L,), jnp.float32)
                for l in range(L):
                    acc2 = acc2 + plsc.load_gather(hist, [b + l])
                return acc2
            accr = gsum(hist_r, base)
            acct = gsum(hist_t, base)
            s_id = j // NV
            pos = (j % NV) * L
            tbuf[s_id, pl.ds(pos, L)] = accr
            tbuf[s_id, pl.ds(64 + pos, L)] = acct
            z = jnp.zeros((L,), jnp.float32)
            for l2 in range(L):
                hist_r[pl.ds((j * L + l2) * L, L)] = z
                hist_t[pl.ds((j * L + l2) * L, L)] = z
            return carry
        lax.fori_loop(0, NBP // L, a2, 0)
        pltpu.sync_copy(tbuf, sh_tot.at[sub])
        plsc.subcore_barrier()

        pltpu.sync_copy(sh_tot.at[:, sub], sbuf)
        cnt_r, cnt_t = [], []
        for v in range(NV):
            ar = jnp.zeros((L,), jnp.float32)
            at_ = jnp.zeros((L,), jnp.float32)
            for t in range(NSUB):
                ar = ar + sbuf[t, pl.ds(v * L, L)]
                at_ = at_ + sbuf[t, pl.ds(64 + v * L, L)]
            cnt_r.append(ar)
            cnt_t.append(at_)
        inc_r, inc_t = [], []
        car = jnp.zeros((), jnp.float32)
        for v in range(NV):
            inc = plsc.cumsum(cnt_r[v]) + car
            inc_r.append(inc)
            car = jnp.max(inc)
        tot_r = car
        car = jnp.zeros((), jnp.float32)
        for v in range(NV):
            inc = plsc.cumsum(cnt_t[v]) + car
            inc_t.append(inc)
            car = jnp.max(inc)
        tot_t = car
        stv = jnp.where(lane == 0, tot_r, jnp.where(lane == 1, tot_t, 0.0))
        cbuf[pl.ds(0, L)] = stv
        pltpu.sync_copy(cbuf, sh_st.at[sub])
        plsc.subcore_barrier()

        pltpu.sync_copy(sh_st, sbuf)
        totr_all = plsc.load_gather(sbuf, [lane, zidx])
        tott_all = plsc.load_gather(sbuf, [lane, zidx + 1])
        before = lane < sub
        pref_r = jnp.sum(jnp.where(before, totr_all, 0.0))
        pref_t = jnp.sum(jnp.where(before, tott_all, 0.0))
        cr_g = [inc_r[v] + pref_r for v in range(NV)]
        cx_g = [cr_g[v] - cnt_r[v] for v in range(NV)]
        for v in range(NV):
            cbuf[pl.ds(v * L, L)] = inc_t[v] + pref_t
        pltpu.sync_copy(cbuf, sh_ct.at[sub])
        plsc.subcore_barrier()
        pltpu.sync_copy(sh_ct, sbuf)
        for j in range(NBP // L):
            ctfull[pl.ds(j * L, L)] = sbuf[j // NV, pl.ds((j % NV) * L, L)]

        for v in range(NV):
            a = q_of(cx_g[v])
            vtop = q_of(cr_g[v])
            gbin = sub * SLICE + v * L + lane
            d = jnp.where(gbin == 0, 0.0, vtop - a)
            cbuf[pl.ds(v * L, L)] = a
            cbuf[pl.ds(64 + v * L, L)] = d
        pltpu.sync_copy(cbuf, sh_ad.at[sub])
        plsc.subcore_barrier()
        pltpu.sync_copy(sh_ad, sbuf)
        for j in range(NBP // L):
            adbuf[0, pl.ds(j * L, L)] = sbuf[j // NV, pl.ds((j % NV) * L, L)]
            adbuf[1, pl.ds(j * L, L)] = sbuf[j // NV, pl.ds(64 + (j % NV) * L, L)]

        start_c(myrow, 0)
        start_c(myrow + CROWS, 1)
        wait_c(0)
        acc = compute_c(0, acc)
        wait_c(1)
        acc = compute_c(1, acc)
        return acc

    acc = lax.fori_loop(0, CPC, chan, jnp.zeros((L,), jnp.float32))
    wid = core * NSUB + sub
    b2[0, 0, pl.ds(0, L)] = acc
    pltpu.sync_copy(b2.at[0, 0, pl.ds(0, L)], out.at[pl.ds(wid * L, L)])


def _pallas_loss(key2, kf2, s2, mv2):
    mesh = plsc.VectorSubcoreMesh(core_axis_name="c", subcore_axis_name="s",
                                  num_cores=NCORES, num_subcores=NSUB)
    return pl.kernel(
        _body,
        out_type=jax.ShapeDtypeStruct((NCORES * NSUB * L,), jnp.float32),
        mesh=mesh,
        compiler_params=pltpu.CompilerParams(needs_layout_passes=False),
        scratch_types=[
            pltpu.VMEM((NBP * L,), jnp.float32),
            pltpu.VMEM((NBP * L,), jnp.float32),
            pltpu.VMEM((2, NBP), jnp.float32),
            pltpu.VMEM((NBP,), jnp.float32),
            pltpu.VMEM((NSUB, 128), jnp.float32),
            pltpu.VMEM((NSUB, 128), jnp.float32),
            pltpu.VMEM((128,), jnp.float32),
            pltpu.VMEM((2, CROWS, W), jnp.int32),
            pltpu.VMEM((2, CROWS, W), jnp.float32),
            pltpu.VMEM((2, CROWS, W), jnp.float32),
            pltpu.VMEM((2, CROWS, W), jnp.float32),
            pltpu.VMEM_SHARED((NSUB, NSUB, 128), jnp.float32),
            pltpu.VMEM_SHARED((NSUB, 128), jnp.float32),
            pltpu.VMEM_SHARED((NSUB, 128), jnp.float32),
            pltpu.VMEM_SHARED((NSUB, 128), jnp.float32),
            pltpu.SemaphoreType.DMA,
            pltpu.SemaphoreType.DMA,
        ],
    )(key2, kf2, s2, mv2)


def kernel(src_img, target_img, src_mask, target_mask, ref_img):
    B, C, h, w = src_img.shape
    n = h * w
    key3, kf3, s3, mv3 = _prep(ref_img, src_mask, target_img,
                               target_mask, src_img)
    out = _pallas_loss(key3.reshape(B * C * h, w), kf3.reshape(B * C * h, w),
                       s3.reshape(B * C * h, w), mv3.reshape(B * C * h, w))
    return jnp.sum(out) / (B * C * n)


# speedup vs baseline: 276.6565x; 1.0470x over previous
import jax
import jax.numpy as jnp
from jax import lax
from jax.experimental import pallas as pl
from jax.experimental.pallas import tpu as pltpu
from jax.experimental.pallas import tpu_sc as plsc

K = 512
L = 16
NCORES = 2
NSUB = 16
NCH = 12
CPC = NCH // NCORES
SLICE = 48
NBP = NSUB * SLICE
NV = SLICE // L
W = 512
TROWS = 32
CROWS = 16
CH = CROWS * W
UNROLL = 8
K_F = float(K)
INVK = 1.0 / K


def _prep_body(ref_b, sm_b, tgt_b, tm_b, src_b, key_b, kf_b, s_b, mv_b):
    m = sm_b[0, 0]
    r = jnp.minimum(jnp.maximum(ref_b[0, 0] * 0.5 + 0.5, 0.0), 1.0) * m
    kf = r * K_F
    kf_b[0] = kf
    mt = tm_b[0, 0]
    t = jnp.minimum(jnp.maximum(tgt_b[0, 0] * 0.5 + 0.5, 0.0), 1.0) * mt
    ir16 = (jnp.ceil(kf) * 16.0).astype(jnp.int32)
    it16 = (jnp.ceil(t * K_F) * 16.0).astype(jnp.int32)
    key_b[0] = ir16 | (it16 << 16)
    s_b[0] = jnp.minimum(jnp.maximum(src_b[0, 0] * 0.5 + 0.5, 0.0), 1.0) * m
    mv_b[0] = m


def _prep(ref4, sm4, tgt4, tm4, src4):
    b, c, h, w = ref4.shape
    rblk = 512
    iblk = (1, 1, rblk, w)
    oblk = (1, rblk, w)
    img_spec = pl.BlockSpec(iblk, lambda i, j: (i // 3, i % 3, j, 0))
    msk_spec = pl.BlockSpec(iblk, lambda i, j: (i // 3, 0, j, 0))
    out_spec = pl.BlockSpec(oblk, lambda i, j: (i, j, 0))
    otype = jax.ShapeDtypeStruct((b * c, h, w), jnp.float32)
    ktype = jax.ShapeDtypeStruct((b * c, h, w), jnp.int32)
    return pl.pallas_call(
        _prep_body,
        grid=(b * c, h // rblk),
        in_specs=[img_spec, msk_spec, img_spec, msk_spec, img_spec],
        out_specs=[out_spec] * 4,
        out_shape=[ktype, otype, otype, otype],
    )(ref4, sm4, tgt4, tm4, src4)


def _body(key2, kf2, s2, mv2, out,
          hist_r, hist_t, adbuf, ctfull, tbuf, sbuf, cbuf,
          bk, b0, b1, b2,
          sh_tot, sh_st, sh_ct, sh_ad,
          sem0, sem1):
    rows_per_ch = key2.shape[0] // NCH
    npix = rows_per_ch * W
    nf = jnp.float32(npix)
    core = lax.axis_index("c")
    sub = lax.axis_index("s")
    lane = lax.iota(jnp.int32, L)
    onesv = jnp.ones((L,), jnp.float32)
    zidx = jnp.zeros((L,), jnp.int32)
    sems = (sem0, sem1)

    def start_a(rb, slot):
        pltpu.make_async_copy(key2.at[pl.ds(rb, CROWS)], bk.at[slot], sems[slot]).start()

    def wait_a(slot):
        pltpu.make_async_copy(key2.at[pl.ds(0, CROWS)], bk.at[slot], sems[slot]).wait()

    def compute_a(slot):
        @plsc.parallel_loop(0, CH // L, unroll=UNROLL)
        def _(v):
            r = v >> 5
            cofs = (v & 31) * L
            w = bk[slot, r, pl.ds(cofs, L)]
            idxr = (w & 0xFFFF) + lane
            idxt = lax.shift_right_logical(w, 16) + lane
            plsc.addupdate_scatter(hist_r, [idxr], onesv)
            plsc.addupdate_scatter(hist_t, [idxt], onesv)

    def start_c(rb, slot):
        pltpu.make_async_copy(kf2.at[pl.ds(rb, CROWS)], b0.at[slot], sems[slot]).start()
        pltpu.make_async_copy(s2.at[pl.ds(rb, CROWS)], b1.at[slot], sems[slot]).start()
        pltpu.make_async_copy(mv2.at[pl.ds(rb, CROWS)], b2.at[slot], sems[slot]).start()

    def wait_c(slot):
        pltpu.make_async_copy(kf2.at[pl.ds(0, CROWS)], b0.at[slot], sems[slot]).wait()
        pltpu.make_async_copy(s2.at[pl.ds(0, CROWS)], b1.at[slot], sems[slot]).wait()
        pltpu.make_async_copy(mv2.at[pl.ds(0, CROWS)], b2.at[slot], sems[slot]).wait()

    def compute_c(slot, acc):
        @plsc.parallel_loop(0, CH // L, unroll=UNROLL, carry=acc)
        def inner(v, acc2):
            r = v >> 5
            cofs = (v & 31) * L
            kf = b0[slot, r, pl.ds(cofs, L)]
            sv = b1[slot, r, pl.ds(cofs, L)]
            mv = b2[slot, r, pl.ds(cofs, L)]
            ki = kf.astype(jnp.int32)
            kif = ki.astype(jnp.float32)
            up = kf > kif
            ki = jnp.where(up, ki + 1, ki)
            frac = (kf - kif) + jnp.where(up, 0.0, 1.0)
            a = plsc.load_gather(adbuf, [zidx, ki])
            dv = plsc.load_gather(adbuf, [zidx + 1, ki])
            matched = a + dv * frac
            diff = sv - mv * matched
            return acc2 + diff * diff
        return inner

    def q_of(p):
        p = jnp.minimum(p, nf - 0.5)
        lo = jnp.zeros((L,), jnp.int32)
        hi = jnp.full((L,), K, jnp.int32)
        for _ in range(10):
            mid = (lo + hi) >> 1
            cm = plsc.load_gather(ctfull, [mid])
            cond = cm > p
            hi = jnp.where(cond, mid, hi)
            lo = jnp.where(cond, lo, mid + 1)
        l = lo
        lm = jnp.maximum(l - 1, 0)
        ctm1 = plsc.load_gather(ctfull, [lm])
        ctm1 = jnp.where(l == 0, 0.0, ctm1)
        cl = plsc.load_gather(ctfull, [l])
        hl = jnp.maximum(cl - ctm1, 1.0)
        v = (l.astype(jnp.float32) - 1.0) * INVK + INVK * (p - ctm1) / hl
        return jnp.where(l == 0, 0.0, v)

    def zero_body(i, carry):
        z = jnp.zeros((L,), jnp.float32)
        hist_r[pl.ds(i * L, L)] = z
        hist_t[pl.ds(i * L, L)] = z
        return carry
    lax.fori_loop(0, NBP, zero_body, 0)

    def chan(ci, acc):
        chrow = (core * CPC + ci) * rows_per_ch
        myrow = chrow + sub * TROWS

        start_a(myrow, 0)
        start_a(myrow + CROWS, 1)
        wait_a(0)
        compute_a(0)
        wait_a(1)
        compute_a(1)

        def a2(j, carry):
            base = (j * L + lane) * L

            def gsum(hist, b):
                acc2 = jnp.zeros((L,), jnp.float32)
                for l in range(L):
                    acc2 = acc2 + plsc.load_gather(hist, [b + l])
                return acc2
            accr = gsum(hist_r, base)
            acct = gsum(hist_t, base)
            s_id = j // NV
            pos = (j % NV) * L
            tbuf[s_id, pl.ds(pos, L)] = accr
            tbuf[s_id, pl.ds(64 + pos, L)] = acct
            z = jnp.zeros((L,), jnp.float32)
            for l2 in range(L):
                hist_r[pl.ds((j * L + l2) * L, L)] = z
                hist_t[pl.ds((j * L + l2) * L, L)] = z
            return carry
        lax.fori_loop(0, NBP // L, a2, 0)
        pltpu.sync_copy(tbuf, sh_tot.at[sub])
        plsc.subcore_barrier()

        pltpu.sync_copy(sh_tot.at[:, sub], sbuf)
        cnt_r, cnt_t = [], []
        for v in range(NV):
            ar = jnp.zeros((L,), jnp.float32)
            at_ = jnp.zeros((L,), jnp.float32)
            for t in range(NSUB):
                ar = ar + sbuf[t, pl.ds(v * L, L)]
                at_ = at_ + sbuf[t, pl.ds(64 + v * L, L)]
            cnt_r.append(ar)
            cnt_t.append(at_)
        inc_r, inc_t = [], []
        car = jnp.zeros((), jnp.float32)
        for v in range(NV):
            inc = plsc.cumsum(cnt_r[v]) + car
            inc_r.append(inc)
            car = jnp.max(inc)
        tot_r = car
        car = jnp.zeros((), jnp.float32)
        for v in range(NV):
            inc = plsc.cumsum(cnt_t[v]) + car
            inc_t.append(inc)
            car = jnp.max(inc)
        tot_t = car
        stv = jnp.where(lane == 0, tot_r, jnp.where(lane == 1, tot_t, 0.0))
        cbuf[pl.ds(0, L)] = stv
        pltpu.sync_copy(cbuf, sh_st.at[sub])
        plsc.subcore_barrier()

        pltpu.sync_copy(sh_st, sbuf)
        totr_all = plsc.load_gather(sbuf, [lane, zidx])
        tott_all = plsc.load_gather(sbuf, [lane, zidx + 1])
        before = lane < sub
        pref_r = jnp.sum(jnp.where(before, totr_all, 0.0))
        pref_t = jnp.sum(jnp.where(before, tott_all, 0.0))
        cr_g = [inc_r[v] + pref_r for v in range(NV)]
        cx_g = [cr_g[v] - cnt_r[v] for v in range(NV)]
        for v in range(NV):
            cbuf[pl.ds(v * L, L)] = inc_t[v] + pref_t
        pltpu.sync_copy(cbuf, sh_ct.at[sub])
        plsc.subcore_barrier()
        pltpu.sync_copy(sh_ct, sbuf)
        for j in range(NBP // L):
            ctfull[pl.ds(j * L, L)] = sbuf[j // NV, pl.ds((j % NV) * L, L)]

        for v in range(NV):
            a = q_of(cx_g[v])
            vtop = q_of(cr_g[v])
            gbin = sub * SLICE + v * L + lane
            d = jnp.where(gbin == 0, 0.0, vtop - a)
            cbuf[pl.ds(v * L, L)] = a
            cbuf[pl.ds(64 + v * L, L)] = d
        pltpu.sync_copy(cbuf, sh_ad.at[sub])
        plsc.subcore_barrier()
        pltpu.sync_copy(sh_ad, sbuf)
        for j in range(NBP // L):
            adbuf[0, pl.ds(j * L, L)] = sbuf[j // NV, pl.ds((j % NV) * L, L)]
            adbuf[1, pl.ds(j * L, L)] = sbuf[j // NV, pl.ds(64 + (j % NV) * L, L)]

        start_c(myrow, 0)
        start_c(myrow + CROWS, 1)
        wait_c(0)
        acc = compute_c(0, acc)
        wait_c(1)
        acc = compute_c(1, acc)
        return acc

    acc = lax.fori_loop(0, CPC, chan, jnp.zeros((L,), jnp.float32))
    wid = core * NSUB + sub
    b2[0, 0, pl.ds(0, L)] = acc
    pltpu.sync_copy(b2.at[0, 0, pl.ds(0, L)], out.at[pl.ds(wid * L, L)])


def _pallas_loss(key2, kf2, s2, mv2):
    mesh = plsc.VectorSubcoreMesh(core_axis_name="c", subcore_axis_name="s",
                                  num_cores=NCORES, num_subcores=NSUB)
    return pl.kernel(
        _body,
        out_type=jax.ShapeDtypeStruct((NCORES * NSUB * L,), jnp.float32),
        mesh=mesh,
        compiler_params=pltpu.CompilerParams(needs_layout_passes=False),
        scratch_types=[
            pltpu.VMEM((NBP * L,), jnp.float32),
            pltpu.VMEM((NBP * L,), jnp.float32),
            pltpu.VMEM((2, NBP), jnp.float32),
            pltpu.VMEM((NBP,), jnp.float32),
            pltpu.VMEM((NSUB, 128), jnp.float32),
            pltpu.VMEM((NSUB, 128), jnp.float32),
            pltpu.VMEM((128,), jnp.float32),
            pltpu.VMEM((2, CROWS, W), jnp.int32),
            pltpu.VMEM((2, CROWS, W), jnp.float32),
            pltpu.VMEM((2, CROWS, W), jnp.float32),
            pltpu.VMEM((2, CROWS, W), jnp.float32),
            pltpu.VMEM_SHARED((NSUB, NSUB, 128), jnp.float32),
            pltpu.VMEM_SHARED((NSUB, 128), jnp.float32),
            pltpu.VMEM_SHARED((NSUB, 128), jnp.float32),
            pltpu.VMEM_SHARED((NSUB, 128), jnp.float32),
            pltpu.SemaphoreType.DMA,
            pltpu.SemaphoreType.DMA,
        ],
    )(key2, kf2, s2, mv2)


def kernel(src_img, target_img, src_mask, target_mask, ref_img):
    B, C, h, w = src_img.shape
    n = h * w
    key3, kf3, s3, mv3 = _prep(ref_img, src_mask, target_img,
                               target_mask, src_img)
    out = _pallas_loss(key3.reshape(B * C * h, w), kf3.reshape(B * C * h, w),
                       s3.reshape(B * C * h, w), mv3.reshape(B * C * h, w))
    return jnp.sum(out) / (B * C * n)
